# Initial kernel scaffold; baseline (speedup 1.0000x reference)
import jax
import jax.numpy as jnp
from jax.experimental import pallas as pl


def kernel(x, pos, t, vars_abc, dt, enc_w1, enc_b1, enc_w2, enc_b2, msg1_w, msg1_b, msg2_w, msg2_b, upd1_w, upd1_b, upd2_w, upd2_b, conv1_w, conv1_b, conv2_w, conv2_b, edge_index, batch):
    raise NotImplementedError("write your pallas kernel here")


# speedup vs baseline: 5.4078x; 5.4078x over previous
import functools

import numpy as np
import jax
import jax.numpy as jnp
from jax import lax
from jax.experimental import pallas as pl
from jax.experimental.pallas import tpu as pltpu
from jax.experimental.pallas import tpu_sc as plsc

N = 10000
E = 320000
ED = 128
NL = 6
NG = 16
TW = 25
T_MAX = 4.0
EPS = 1e-5

NC = 2
NS = 16
NW = NC * NS
EPW = E // NW
R = 80
J = EPW // R
NCHUNK = E // R
NPAD = 10240
RPT = NPAD // NS

BN = 1000
GN = N // BN
BE = 2000
GE = E // BE

_SC_MESH = plsc.VectorSubcoreMesh(core_axis_name="c", subcore_axis_name="s")


def _silu(v):
    return v * jax.nn.sigmoid(v)


def _sc_gather_body(a_hbm, b_hbm, dst_hbm, src_hbm, outa_hbm, outb_hbm,
                    idxd_v, idxs_v, rows_a, rows_b, sem_a, sem_b):
    c = lax.axis_index("c")
    s = lax.axis_index("s")
    wid = s * NC + c
    pltpu.sync_copy(dst_hbm.at[wid], idxd_v)
    pltpu.sync_copy(src_hbm.at[wid], idxs_v)

    def step(j, carry):
        ck = wid * J + j
        cpa = pltpu.async_copy(a_hbm.at[idxd_v.at[j]], rows_a, sem_a)
        cpb = pltpu.async_copy(b_hbm.at[idxs_v.at[j]], rows_b, sem_b)
        cpa.wait()
        pltpu.sync_copy(rows_a, outa_hbm.at[ck])
        cpb.wait()
        pltpu.sync_copy(rows_b, outb_hbm.at[ck])
        return carry

    lax.fori_loop(0, J, step, 0)


def _sc_gather(a, b, dst3d, src3d):
    f = pl.kernel(
        _sc_gather_body,
        out_type=(jax.ShapeDtypeStruct((NCHUNK, R, ED), jnp.float32),
                  jax.ShapeDtypeStruct((NCHUNK, R, ED), jnp.float32)),
        mesh=_SC_MESH,
        scratch_types=[
            pltpu.VMEM((J, R), jnp.int32),
            pltpu.VMEM((J, R), jnp.int32),
            pltpu.VMEM((R, ED), jnp.float32),
            pltpu.VMEM((R, ED), jnp.float32),
            pltpu.SemaphoreType.DMA,
            pltpu.SemaphoreType.DMA,
        ],
    )
    return f(a, b, dst3d, src3d)


def _sc_scatter_body(m_hbm, dst_hbm, out_hbm, idx_v, rows_v, acc_sh):
    c = lax.axis_index("c")
    s = lax.axis_index("s")
    wid = s * NC + c

    def zrow(r, carry):
        for jj in range(ED // 16):
            rows_v[r, pl.ds(jj * 16, 16)] = jnp.zeros((16,), jnp.float32)
        return carry

    lax.fori_loop(0, R, zrow, 0)
    for q in range(RPT // R):
        pltpu.sync_copy(rows_v, acc_sh.at[pl.ds(s * RPT + q * R, R)])
    pltpu.sync_copy(dst_hbm.at[wid], idx_v)
    plsc.subcore_barrier()

    def step(j, carry):
        pltpu.sync_copy(m_hbm.at[wid * J + j], rows_v)
        pltpu.sync_copy(rows_v, acc_sh.at[idx_v.at[j]], add=True)
        return carry

    lax.fori_loop(0, J, step, 0)
    plsc.subcore_barrier()
    pltpu.sync_copy(acc_sh.at[pl.ds(s * RPT, RPT)],
                    out_hbm.at[c, pl.ds(s * RPT, RPT)])


def _sc_scatter(m, dst3d):
    f = pl.kernel(
        _sc_scatter_body,
        out_type=jax.ShapeDtypeStruct((NC, NPAD, ED), jnp.float32),
        mesh=_SC_MESH,
        scratch_types=[
            pltpu.VMEM((J, R), jnp.int32),
            pltpu.VMEM((R, ED), jnp.float32),
            pltpu.VMEM_SHARED((NPAD, ED), jnp.float32),
        ],
    )
    return f(m, dst3d)


def _sc_deg_body(dst_hbm, out_hbm, idx_v, ones_v, z_v, acc_sh):
    c = lax.axis_index("c")
    s = lax.axis_index("s")
    wid = s * NC + c

    def frow(r, carry):
        for jj in range(ED // 16):
            ones_v[r, pl.ds(jj * 16, 16)] = jnp.ones((16,), jnp.float32)
            z_v[r, pl.ds(jj * 16, 16)] = jnp.zeros((16,), jnp.float32)
        return carry

    lax.fori_loop(0, R, frow, 0)
    for q in range(RPT // R):
        pltpu.sync_copy(z_v, acc_sh.at[pl.ds(s * RPT + q * R, R)])
    pltpu.sync_copy(dst_hbm.at[wid], idx_v)
    plsc.subcore_barrier()

    def step(j, carry):
        pltpu.sync_copy(ones_v, acc_sh.at[idx_v.at[j]], add=True)
        return carry

    lax.fori_loop(0, J, step, 0)
    plsc.subcore_barrier()
    pltpu.sync_copy(acc_sh.at[pl.ds(s * RPT, RPT)],
                    out_hbm.at[c, pl.ds(s * RPT, RPT)])


def _sc_deg(dst3d):
    f = pl.kernel(
        _sc_deg_body,
        out_type=jax.ShapeDtypeStruct((NC, NPAD, ED), jnp.float32),
        mesh=_SC_MESH,
        scratch_types=[
            pltpu.VMEM((J, R), jnp.int32),
            pltpu.VMEM((R, ED), jnp.float32),
            pltpu.VMEM((R, ED), jnp.float32),
            pltpu.VMEM_SHARED((NPAD, ED), jnp.float32),
        ],
    )
    return f(dst3d)


def _posmax_body(pos_ref, out_ref):
    out_ref[0, 0] = jnp.max(pos_ref[...])


def _posmax(pos_pad):
    return pl.pallas_call(
        _posmax_body,
        out_shape=jax.ShapeDtypeStruct((1, 1), jnp.float32),
        in_specs=[pl.BlockSpec(pos_pad.shape, lambda: (0, 0))],
        out_specs=pl.BlockSpec((1, 1), lambda: (0, 0),
                               memory_space=pltpu.SMEM),
    )(pos_pad)


def _enc_body(posmax_ref, inp_ref, batch_ref, ew1t_ref, eb1_ref, ew2t_ref,
              eb2_ref, wsd_ref, bsd_ref, wsn_ref, wsu_ref, bsu_ref,
              whd0t_ref, whs0t_ref,
              h0_ref, a0_ref, b0_ref, sd_ref, ssn_ref, su_ref, cb_ref):
    col = lax.broadcasted_iota(jnp.int32, (1, 30), 1)
    inv_pm = 1.0 / posmax_ref[0, 0]
    scale = jnp.where(col == 25, inv_pm,
                      jnp.where(col == 26, 1.0 / T_MAX, 1.0))
    xb = inp_ref[...] * scale
    h1 = _silu(jnp.dot(xb, ew1t_ref[...],
                       preferred_element_type=jnp.float32) + eb1_ref[...])
    h0 = _silu(jnp.dot(h1, ew2t_ref[...],
                       preferred_element_type=jnp.float32) + eb2_ref[...])
    sd = jnp.dot(xb, wsd_ref[...],
                 preferred_element_type=jnp.float32) + bsd_ref[...]
    ssn = jnp.dot(xb, wsn_ref[...], preferred_element_type=jnp.float32)
    su = jnp.dot(xb, wsu_ref[...],
                 preferred_element_type=jnp.float32) + bsu_ref[...]
    h0_ref[...] = h0
    a0_ref[...] = jnp.dot(h0, whd0t_ref[...],
                          preferred_element_type=jnp.float32) + sd[:, :ED]
    b0_ref[...] = jnp.dot(h0, whs0t_ref[...],
                          preferred_element_type=jnp.float32) + ssn[:, :ED]
    sd_ref[...] = sd
    ssn_ref[...] = ssn
    su_ref[...] = su
    mask = (batch_ref[...] ==
            lax.broadcasted_iota(jnp.int32, (1, NG), 1)).astype(jnp.float32)
    part = jnp.sum(mask, axis=0, keepdims=True)

    @pl.when(pl.program_id(0) == 0)
    def _():
        cb_ref[...] = jnp.zeros_like(cb_ref)

    cb_ref[...] += part


def _encoder(posmax, inp30, batch2d, ew1t, eb1, ew2t, eb2, wsd, bsd, wsn,
             wsu, bsu, whd0t, whs0t):
    SD = NL * ED
    full = lambda shp: pl.BlockSpec(shp, lambda i: (0, 0))
    return pl.pallas_call(
        _enc_body,
        grid=(GN,),
        out_shape=(
            jax.ShapeDtypeStruct((N, ED), jnp.float32),
            jax.ShapeDtypeStruct((N, ED), jnp.float32),
            jax.ShapeDtypeStruct((N, ED), jnp.float32),
            jax.ShapeDtypeStruct((N, SD), jnp.float32),
            jax.ShapeDtypeStruct((N, SD), jnp.float32),
            jax.ShapeDtypeStruct((N, SD), jnp.float32),
            jax.ShapeDtypeStruct((1, NG), jnp.float32),
        ),
        in_specs=[
            pl.BlockSpec(memory_space=pltpu.SMEM),
            pl.BlockSpec((BN, 30), lambda i: (i, 0)),
            pl.BlockSpec((BN, 1), lambda i: (i, 0)),
            full((30, ED)), full((1, ED)), full((ED, ED)), full((1, ED)),
            full((30, SD)), full((1, SD)), full((30, SD)), full((30, SD)),
            full((1, SD)), full((ED, ED)), full((ED, ED)),
        ],
        out_specs=(
            pl.BlockSpec((BN, ED), lambda i: (i, 0)),
            pl.BlockSpec((BN, ED), lambda i: (i, 0)),
            pl.BlockSpec((BN, ED), lambda i: (i, 0)),
            pl.BlockSpec((BN, SD), lambda i: (i, 0)),
            pl.BlockSpec((BN, SD), lambda i: (i, 0)),
            pl.BlockSpec((BN, SD), lambda i: (i, 0)),
            pl.BlockSpec((1, NG), lambda i: (0, 0)),
        ),
    )(posmax, inp30, batch2d, ew1t, eb1, ew2t, eb2, wsd, bsd, wsn, wsu,
      bsu, whd0t, whs0t)


def _edge_body(pa_ref, pb_ref, w2t_ref, b2_ref, m_ref):
    bc = BE // R
    pa = pa_ref[...].reshape(BE, ED)
    pb = pb_ref[...].reshape(BE, ED)
    sv = _silu(pa + pb)
    mv = _silu(jnp.dot(sv, w2t_ref[...],
                       preferred_element_type=jnp.float32) + b2_ref[...])
    m_ref[...] = mv.reshape(bc, R, ED)


def _edge_mlp(pa, pb, w2t, b2):
    bc = BE // R
    return pl.pallas_call(
        _edge_body,
        grid=(GE,),
        out_shape=jax.ShapeDtypeStruct((NCHUNK, R, ED), jnp.float32),
        in_specs=[
            pl.BlockSpec((bc, R, ED), lambda i: (i, 0, 0)),
            pl.BlockSpec((bc, R, ED), lambda i: (i, 0, 0)),
            pl.BlockSpec((ED, ED), lambda i: (0, 0)),
            pl.BlockSpec((1, ED), lambda i: (0, 0)),
        ],
        out_specs=pl.BlockSpec((bc, R, ED), lambda i: (i, 0, 0)),
    )(pa, pb, w2t, b2)


def _upd_body(h_ref, p_ref, deg_ref, su_ref, uht_ref, uat_ref, u2t_ref,
              ub2_ref, batch_ref, hn_ref, gsum_ref, gsq_ref):
    psum = p_ref[0] + p_ref[1]
    degv = jnp.maximum(deg_ref[0, :, 0:1] + deg_ref[1, :, 0:1], 1.0)
    agg = psum * (1.0 / degv)
    u1 = _silu(jnp.dot(h_ref[...], uht_ref[...],
                       preferred_element_type=jnp.float32)
               + jnp.dot(agg, uat_ref[...],
                         preferred_element_type=jnp.float32)
               + su_ref[...])
    up = _silu(jnp.dot(u1, u2t_ref[...],
                       preferred_element_type=jnp.float32) + ub2_ref[...])
    hn = h_ref[...] + up
    hn_ref[...] = hn
    mask = (batch_ref[...] ==
            lax.broadcasted_iota(jnp.int32, (1, NG), 1)).astype(jnp.float32)
    gs = lax.dot_general(mask, hn, (((0,), (0,)), ((), ())),
                         preferred_element_type=jnp.float32)
    gq = lax.dot_general(mask, hn * hn, (((0,), (0,)), ((), ())),
                         preferred_element_type=jnp.float32)

    @pl.when(pl.program_id(0) == 0)
    def _():
        gsum_ref[...] = jnp.zeros_like(gsum_ref)
        gsq_ref[...] = jnp.zeros_like(gsq_ref)

    gsum_ref[...] += gs
    gsq_ref[...] += gq


def _update(h, partials, deg2, su_all, layer, uht, uat, u2t, ub2, batch2d):
    return pl.pallas_call(
        _upd_body,
        grid=(GN,),
        out_shape=(
            jax.ShapeDtypeStruct((N, ED), jnp.float32),
            jax.ShapeDtypeStruct((NG, ED), jnp.float32),
            jax.ShapeDtypeStruct((NG, ED), jnp.float32),
        ),
        in_specs=[
            pl.BlockSpec((BN, ED), lambda i: (i, 0)),
            pl.BlockSpec((NC, BN, ED), lambda i: (0, i, 0)),
            pl.BlockSpec((NC, BN, ED), lambda i: (0, i, 0)),
            pl.BlockSpec((BN, ED), lambda i, L=layer: (i, L)),
            pl.BlockSpec((ED, ED), lambda i: (0, 0)),
            pl.BlockSpec((ED, ED), lambda i: (0, 0)),
            pl.BlockSpec((ED, ED), lambda i: (0, 0)),
            pl.BlockSpec((1, ED), lambda i: (0, 0)),
            pl.BlockSpec((BN, 1), lambda i: (i, 0)),
        ],
        out_specs=(
            pl.BlockSpec((BN, ED), lambda i: (i, 0)),
            pl.BlockSpec((NG, ED), lambda i: (0, 0)),
            pl.BlockSpec((NG, ED), lambda i: (0, 0)),
        ),
    )(h, partials, deg2, su_all, uht, uat, u2t, ub2, batch2d)


def _norm_body_proj(hn_ref, gsum_ref, gsq_ref, cb_ref, batch_ref, whdt_ref,
                    whst_ref, sd_ref, ssn_ref, h_ref, a_ref, b_ref):
    hb = _norm_common(hn_ref, gsum_ref, gsq_ref, cb_ref, batch_ref)
    h_ref[...] = hb
    a_ref[...] = jnp.dot(hb, whdt_ref[...],
                         preferred_element_type=jnp.float32) + sd_ref[...]
    b_ref[...] = jnp.dot(hb, whst_ref[...],
                         preferred_element_type=jnp.float32) + ssn_ref[...]


def _norm_body_last(hn_ref, gsum_ref, gsq_ref, cb_ref, batch_ref, h_ref):
    h_ref[...] = _norm_common(hn_ref, gsum_ref, gsq_ref, cb_ref, batch_ref)


def _norm_common(hn_ref, gsum_ref, gsq_ref, cb_ref, batch_ref):
    cbm = jnp.maximum(cb_ref[...], 1.0)
    mask = (batch_ref[...] ==
            lax.broadcasted_iota(jnp.int32, (1, NG), 1)).astype(jnp.float32)
    maskc = mask * (1.0 / cbm)
    meanr = jnp.dot(maskc, gsum_ref[...], preferred_element_type=jnp.float32)
    eh2r = jnp.dot(maskc, gsq_ref[...], preferred_element_type=jnp.float32)
    varr = jnp.maximum(eh2r - meanr * meanr, 0.0)
    return (hn_ref[...] - meanr) * lax.rsqrt(varr + EPS)


def _norm_proj(hn, gsum, gsq, cb, batch2d, whdt, whst, sd_all, ssn_all,
               layer_next):
    return pl.pallas_call(
        _norm_body_proj,
        grid=(GN,),
        out_shape=(
            jax.ShapeDtypeStruct((N, ED), jnp.float32),
            jax.ShapeDtypeStruct((N, ED), jnp.float32),
            jax.ShapeDtypeStruct((N, ED), jnp.float32),
        ),
        in_specs=[
            pl.BlockSpec((BN, ED), lambda i: (i, 0)),
            pl.BlockSpec((NG, ED), lambda i: (0, 0)),
            pl.BlockSpec((NG, ED), lambda i: (0, 0)),
            pl.BlockSpec((1, NG), lambda i: (0, 0)),
            pl.BlockSpec((BN, 1), lambda i: (i, 0)),
            pl.BlockSpec((ED, ED), lambda i: (0, 0)),
            pl.BlockSpec((ED, ED), lambda i: (0, 0)),
            pl.BlockSpec((BN, ED), lambda i, L=layer_next: (i, L)),
            pl.BlockSpec((BN, ED), lambda i, L=layer_next: (i, L)),
        ],
        out_specs=(
            pl.BlockSpec((BN, ED), lambda i: (i, 0)),
            pl.BlockSpec((BN, ED), lambda i: (i, 0)),
            pl.BlockSpec((BN, ED), lambda i: (i, 0)),
        ),
    )(hn, gsum, gsq, cb, batch2d, whdt, whst, sd_all, ssn_all)


def _norm_last(hn, gsum, gsq, cb, batch2d):
    return pl.pallas_call(
        _norm_body_last,
        grid=(GN,),
        out_shape=jax.ShapeDtypeStruct((N, ED), jnp.float32),
        in_specs=[
            pl.BlockSpec((BN, ED), lambda i: (i, 0)),
            pl.BlockSpec((NG, ED), lambda i: (0, 0)),
            pl.BlockSpec((NG, ED), lambda i: (0, 0)),
            pl.BlockSpec((1, NG), lambda i: (0, 0)),
            pl.BlockSpec((BN, 1), lambda i: (i, 0)),
        ],
        out_specs=pl.BlockSpec((BN, ED), lambda i: (i, 0)),
    )(hn, gsum, gsq, cb, batch2d)


def _dec_body(h_ref, x_ref, dt_ref, w1d_ref, b1d_ref, w2d_ref, b2s_ref,
              out_ref):
    z1 = _silu(jnp.dot(h_ref[...], w1d_ref[...],
                       preferred_element_type=jnp.float32) + b1d_ref[...])
    z2 = jnp.dot(z1, w2d_ref[...],
                 preferred_element_type=jnp.float32) + b2s_ref[0, 0]
    steps = (lax.broadcasted_iota(jnp.int32, (1, TW), 1) + 1
             ).astype(jnp.float32)
    dtv = dt_ref[0, 0] * steps
    out_ref[...] = x_ref[:, TW - 1:TW] + dtv * z2


def _decoder(h, x, dt2d, w1d, b1d, w2d, b2s):
    return pl.pallas_call(
        _dec_body,
        grid=(GN,),
        out_shape=jax.ShapeDtypeStruct((N, TW), jnp.float32),
        in_specs=[
            pl.BlockSpec((BN, ED), lambda i: (i, 0)),
            pl.BlockSpec((BN, TW), lambda i: (i, 0)),
            pl.BlockSpec(memory_space=pltpu.SMEM),
            pl.BlockSpec((ED, 8 * 38), lambda i: (0, 0)),
            pl.BlockSpec((1, 8 * 38), lambda i: (0, 0)),
            pl.BlockSpec((8 * 38, TW), lambda i: (0, 0)),
            pl.BlockSpec(memory_space=pltpu.SMEM),
        ],
        out_specs=pl.BlockSpec((BN, TW), lambda i: (i, 0)),
    )(h, x, dt2d, w1d, b1d, w2d, b2s)


def _densify_convs(conv1_w, conv1_b, conv2_w, conv2_b):
    o_i, p_i, k_i = np.meshgrid(np.arange(8), np.arange(38), np.arange(16),
                                indexing="ij")
    rows1 = (3 * p_i + k_i).reshape(-1)
    cols1 = (o_i * 38 + p_i).reshape(-1)
    w1d = jnp.zeros((ED, 8 * 38), jnp.float32).at[rows1, cols1].set(
        conv1_w[o_i.reshape(-1), 0, k_i.reshape(-1)])
    b1d = jnp.repeat(conv1_b, 38).reshape(1, 8 * 38)
    o_i, q_i, d_i = np.meshgrid(np.arange(8), np.arange(TW), np.arange(14),
                                indexing="ij")
    rows2 = (o_i * 38 + q_i + d_i).reshape(-1)
    cols2 = q_i.reshape(-1)
    w2d = jnp.zeros((8 * 38, TW), jnp.float32).at[rows2, cols2].set(
        conv2_w[0, o_i.reshape(-1), d_i.reshape(-1)])
    b2s = conv2_b.reshape(1, 1)
    return w1d, b1d, w2d, b2s


def kernel(x, pos, t, vars_abc, dt, enc_w1, enc_b1, enc_w2, enc_b2,
           msg1_w, msg1_b, msg2_w, msg2_b, upd1_w, upd1_b, upd2_w, upd2_b,
           conv1_w, conv1_b, conv2_w, conv2_b, edge_index, batch):
    f32 = jnp.float32
    SD = NL * ED

    inp30 = jnp.concatenate([x, pos, t, vars_abc], axis=-1)
    batch2d = batch.reshape(N, 1).astype(jnp.int32)
    src3d = edge_index[0].astype(jnp.int32).reshape(NW, J, R)
    dst3d = edge_index[1].astype(jnp.int32).reshape(NW, J, R)
    pos_pad = jnp.concatenate(
        [pos.reshape(-1), jnp.zeros((240,), f32)]).reshape(80, 128)

    whd_t = jnp.transpose(msg1_w[:, :, :ED], (0, 2, 1))
    whs_t = jnp.transpose(msg1_w[:, :, ED:2 * ED], (0, 2, 1))
    wx = msg1_w[:, :, 2 * ED:2 * ED + TW]
    wp = msg1_w[:, :, 2 * ED + TW:2 * ED + TW + 1]
    wv = msg1_w[:, :, 2 * ED + TW + 1:]
    wsd = jnp.concatenate([wx, wp, wv], axis=2)
    wsd_t = wsd.reshape(SD, 30).T
    bsd = msg1_b.reshape(1, SD)
    wsn = -jnp.concatenate([wx, wp, jnp.zeros((NL, ED, 4), f32)], axis=2)
    wsn_t = wsn.reshape(SD, 30).T
    uv = upd1_w[:, :, 2 * ED:]
    wsu = jnp.concatenate([jnp.zeros((NL, ED, 26), f32), uv], axis=2)
    wsu_t = wsu.reshape(SD, 30).T
    bsu = upd1_b.reshape(1, SD)
    uh_t = jnp.transpose(upd1_w[:, :, :ED], (0, 2, 1))
    ua_t = jnp.transpose(upd1_w[:, :, ED:2 * ED], (0, 2, 1))
    w2_t = jnp.transpose(msg2_w, (0, 2, 1))
    u2_t = jnp.transpose(upd2_w, (0, 2, 1))
    ew1t = enc_w1.T
    ew2t = enc_w2.T
    eb1 = enc_b1.reshape(1, ED)
    eb2 = enc_b2.reshape(1, ED)
    mb2 = msg2_b.reshape(NL, 1, ED)
    ub2 = upd2_b.reshape(NL, 1, ED)
    w1d, b1d, w2d, b2s = _densify_convs(conv1_w, conv1_b, conv2_w, conv2_b)
    dt2d = dt.reshape(1, 1)

    posmax = _posmax(pos_pad)
    h, a, b, sd_all, ssn_all, su_all, cb = _encoder(
        posmax, inp30, batch2d, ew1t, eb1, ew2t, eb2, wsd_t, bsd, wsn_t,
        wsu_t, bsu, whd_t[0].copy(), whs_t[0].copy())
    deg2 = _sc_deg(dst3d)

    for l in range(NL):
        pa, pb = _sc_gather(a, b, dst3d, src3d)
        m = _edge_mlp(pa, pb, w2_t[l], mb2[l])
        partials = _sc_scatter(m, dst3d)
        hn, gsum, gsq = _update(h, partials, deg2, su_all, l, uh_t[l],
                                ua_t[l], u2_t[l], ub2[l], batch2d)
        if l < NL - 1:
            h, a, b = _norm_proj(hn, gsum, gsq, cb, batch2d, whd_t[l + 1],
                                 whs_t[l + 1], sd_all, ssn_all, l + 1)
        else:
            h = _norm_last(hn, gsum, gsq, cb, batch2d)

    return _decoder(h, x, dt2d, w1d, b1d, w2d, b2s)


# speedup vs baseline: 7.7008x; 1.4240x over previous
import functools

import numpy as np
import jax
import jax.numpy as jnp
from jax import lax
from jax.experimental import pallas as pl
from jax.experimental.pallas import tpu as pltpu
from jax.experimental.pallas import tpu_sc as plsc

N = 10000
E = 320000
ED = 128
NL = 6
NG = 16
TW = 25
T_MAX = 4.0
EPS = 1e-5

NC = 2
NS = 16
NW = NC * NS
EPW = E // NW
R = 80
J = EPW // R
NCHUNK = E // R
NPAD = 10240
RPT = NPAD // NS

BN = 1000
GN = N // BN
BE = 2000
GE = E // BE

_SC_MESH = plsc.VectorSubcoreMesh(core_axis_name="c", subcore_axis_name="s")


def _silu(v):
    return v * jax.nn.sigmoid(v)


def _add_rows(dst_v, src_v):
    def row(r, carry):
        for jj in range(ED // 16):
            sl = pl.ds(jj * 16, 16)
            dst_v[r, sl] += src_v[r, sl]
        return carry

    lax.fori_loop(0, R, row, 0)


def _sc_gather_body(a_hbm, b_hbm, dst_hbm, src_hbm, out_hbm,
                    idxd_v, idxs_v, a0, b0, a1, b1, sa0, sb0, sa1, sb1):
    c = lax.axis_index("c")
    s = lax.axis_index("s")
    wid = s * NC + c
    pltpu.sync_copy(dst_hbm.at[wid], idxd_v)
    pltpu.sync_copy(src_hbm.at[wid], idxs_v)
    base = wid * J

    def fire(ck_local, ra, rb, sa, sb):
        cpa = pltpu.async_copy(a_hbm.at[idxd_v.at[ck_local]], ra, sa)
        cpb = pltpu.async_copy(b_hbm.at[idxs_v.at[ck_local]], rb, sb)
        return cpa, cpb

    def drain(ck_local, ra, rb, sa, sb):
        pltpu.make_async_copy(a_hbm.at[idxd_v.at[ck_local]], ra, sa).wait()
        pltpu.make_async_copy(b_hbm.at[idxs_v.at[ck_local]], rb, sb).wait()
        _add_rows(ra, rb)
        pltpu.sync_copy(ra, out_hbm.at[base + ck_local])

    fire(0, a0, b0, sa0, sb0)

    def step(i, carry):
        fire(2 * i + 1, a1, b1, sa1, sb1)
        drain(2 * i, a0, b0, sa0, sb0)
        fire(2 * i + 2, a0, b0, sa0, sb0)
        drain(2 * i + 1, a1, b1, sa1, sb1)
        return carry

    lax.fori_loop(0, (J - 1) // 2, step, 0)
    drain(J - 1, a0, b0, sa0, sb0)


def _sc_gather(a, b, dst3d, src3d):
    f = pl.kernel(
        _sc_gather_body,
        out_type=jax.ShapeDtypeStruct((NCHUNK, R, ED), jnp.float32),
        mesh=_SC_MESH,
        scratch_types=[
            pltpu.VMEM((J, R), jnp.int32),
            pltpu.VMEM((J, R), jnp.int32),
            pltpu.VMEM((R, ED), jnp.float32),
            pltpu.VMEM((R, ED), jnp.float32),
            pltpu.VMEM((R, ED), jnp.float32),
            pltpu.VMEM((R, ED), jnp.float32),
            pltpu.SemaphoreType.DMA,
            pltpu.SemaphoreType.DMA,
            pltpu.SemaphoreType.DMA,
            pltpu.SemaphoreType.DMA,
        ],
    )
    return f(a, b, dst3d, src3d)


def _sc_scatter_body(m_hbm, dst_hbm, out_hbm, idx_v, m0_v, m1_v, acc_sh,
                     s0, s1):
    c = lax.axis_index("c")
    s = lax.axis_index("s")
    wid = s * NC + c

    def zrow(r, carry):
        for jj in range(ED // 16):
            m0_v[r, pl.ds(jj * 16, 16)] = jnp.zeros((16,), jnp.float32)
        return carry

    lax.fori_loop(0, R, zrow, 0)
    for q in range(RPT // R):
        pltpu.sync_copy(m0_v, acc_sh.at[pl.ds(s * RPT + q * R, R)])
    pltpu.sync_copy(dst_hbm.at[wid], idx_v)
    plsc.subcore_barrier()
    base = wid * J

    pltpu.async_copy(m_hbm.at[base], m0_v, s0)

    def step(i, carry):
        pltpu.async_copy(m_hbm.at[base + 2 * i + 1], m1_v, s1)
        pltpu.make_async_copy(m_hbm.at[base + 2 * i], m0_v, s0).wait()
        pltpu.sync_copy(m0_v, acc_sh.at[idx_v.at[2 * i]], add=True)
        pltpu.async_copy(m_hbm.at[base + 2 * i + 2], m0_v, s0)
        pltpu.make_async_copy(m_hbm.at[base + 2 * i + 1], m1_v, s1).wait()
        pltpu.sync_copy(m1_v, acc_sh.at[idx_v.at[2 * i + 1]], add=True)
        return carry

    lax.fori_loop(0, (J - 1) // 2, step, 0)
    pltpu.make_async_copy(m_hbm.at[base + J - 1], m0_v, s0).wait()
    pltpu.sync_copy(m0_v, acc_sh.at[idx_v.at[J - 1]], add=True)
    plsc.subcore_barrier()
    pltpu.sync_copy(acc_sh.at[pl.ds(s * RPT, RPT)],
                    out_hbm.at[c, pl.ds(s * RPT, RPT)])


def _sc_scatter(m, dst3d):
    f = pl.kernel(
        _sc_scatter_body,
        out_type=jax.ShapeDtypeStruct((NC, NPAD, ED), jnp.float32),
        mesh=_SC_MESH,
        scratch_types=[
            pltpu.VMEM((J, R), jnp.int32),
            pltpu.VMEM((R, ED), jnp.float32),
            pltpu.VMEM((R, ED), jnp.float32),
            pltpu.VMEM_SHARED((NPAD, ED), jnp.float32),
            pltpu.SemaphoreType.DMA,
            pltpu.SemaphoreType.DMA,
        ],
    )
    return f(m, dst3d)


def _sc_deg_body(dst_hbm, out_hbm, idx_v, ones_v, z_v, acc_sh):
    c = lax.axis_index("c")
    s = lax.axis_index("s")
    wid = s * NC + c

    def frow(r, carry):
        for jj in range(ED // 16):
            ones_v[r, pl.ds(jj * 16, 16)] = jnp.ones((16,), jnp.float32)
            z_v[r, pl.ds(jj * 16, 16)] = jnp.zeros((16,), jnp.float32)
        return carry

    lax.fori_loop(0, R, frow, 0)
    for q in range(RPT // R):
        pltpu.sync_copy(z_v, acc_sh.at[pl.ds(s * RPT + q * R, R)])
    pltpu.sync_copy(dst_hbm.at[wid], idx_v)
    plsc.subcore_barrier()

    def step(j, carry):
        pltpu.sync_copy(ones_v, acc_sh.at[idx_v.at[j]], add=True)
        return carry

    lax.fori_loop(0, J, step, 0)
    plsc.subcore_barrier()
    pltpu.sync_copy(acc_sh.at[pl.ds(s * RPT, RPT)],
                    out_hbm.at[c, pl.ds(s * RPT, RPT)])


def _sc_deg(dst3d):
    f = pl.kernel(
        _sc_deg_body,
        out_type=jax.ShapeDtypeStruct((NC, NPAD, ED), jnp.float32),
        mesh=_SC_MESH,
        scratch_types=[
            pltpu.VMEM((J, R), jnp.int32),
            pltpu.VMEM((R, ED), jnp.float32),
            pltpu.VMEM((R, ED), jnp.float32),
            pltpu.VMEM_SHARED((NPAD, ED), jnp.float32),
        ],
    )
    return f(dst3d)


def _posmax_body(pos_ref, out_ref):
    out_ref[0, 0] = jnp.max(pos_ref[...])


def _posmax(pos_pad):
    return pl.pallas_call(
        _posmax_body,
        out_shape=jax.ShapeDtypeStruct((1, 1), jnp.float32),
        in_specs=[pl.BlockSpec(pos_pad.shape, lambda: (0, 0))],
        out_specs=pl.BlockSpec((1, 1), lambda: (0, 0),
                               memory_space=pltpu.SMEM),
    )(pos_pad)


def _enc_body(posmax_ref, inp_ref, batch_ref, ew1t_ref, eb1_ref, ew2t_ref,
              eb2_ref, wsd_ref, bsd_ref, wsn_ref, wsu_ref, bsu_ref,
              whd0t_ref, whs0t_ref,
              h0_ref, a0_ref, b0_ref, sd_ref, ssn_ref, su_ref, cb_ref):
    col = lax.broadcasted_iota(jnp.int32, (1, 30), 1)
    inv_pm = 1.0 / posmax_ref[0, 0]
    scale = jnp.where(col == 25, inv_pm,
                      jnp.where(col == 26, 1.0 / T_MAX, 1.0))
    xb = inp_ref[...] * scale
    h1 = _silu(jnp.dot(xb, ew1t_ref[...],
                       preferred_element_type=jnp.float32) + eb1_ref[...])
    h0 = _silu(jnp.dot(h1, ew2t_ref[...],
                       preferred_element_type=jnp.float32) + eb2_ref[...])
    sd = jnp.dot(xb, wsd_ref[...],
                 preferred_element_type=jnp.float32) + bsd_ref[...]
    ssn = jnp.dot(xb, wsn_ref[...], preferred_element_type=jnp.float32)
    su = jnp.dot(xb, wsu_ref[...],
                 preferred_element_type=jnp.float32) + bsu_ref[...]
    h0_ref[...] = h0
    a0_ref[...] = jnp.dot(h0, whd0t_ref[...],
                          preferred_element_type=jnp.float32) + sd[:, :ED]
    b0_ref[...] = jnp.dot(h0, whs0t_ref[...],
                          preferred_element_type=jnp.float32) + ssn[:, :ED]
    sd_ref[...] = sd
    ssn_ref[...] = ssn
    su_ref[...] = su
    mask = (batch_ref[...] ==
            lax.broadcasted_iota(jnp.int32, (1, NG), 1)).astype(jnp.float32)
    part = jnp.sum(mask, axis=0, keepdims=True)

    @pl.when(pl.program_id(0) == 0)
    def _():
        cb_ref[...] = jnp.zeros_like(cb_ref)

    cb_ref[...] += part


def _encoder(posmax, inp30, batch2d, ew1t, eb1, ew2t, eb2, wsd, bsd, wsn,
             wsu, bsu, whd0t, whs0t):
    SD = NL * ED
    full = lambda shp: pl.BlockSpec(shp, lambda i: (0, 0))
    return pl.pallas_call(
        _enc_body,
        grid=(GN,),
        out_shape=(
            jax.ShapeDtypeStruct((N, ED), jnp.float32),
            jax.ShapeDtypeStruct((N, ED), jnp.float32),
            jax.ShapeDtypeStruct((N, ED), jnp.float32),
            jax.ShapeDtypeStruct((N, SD), jnp.float32),
            jax.ShapeDtypeStruct((N, SD), jnp.float32),
            jax.ShapeDtypeStruct((N, SD), jnp.float32),
            jax.ShapeDtypeStruct((1, NG), jnp.float32),
        ),
        in_specs=[
            pl.BlockSpec(memory_space=pltpu.SMEM),
            pl.BlockSpec((BN, 30), lambda i: (i, 0)),
            pl.BlockSpec((BN, 1), lambda i: (i, 0)),
            full((30, ED)), full((1, ED)), full((ED, ED)), full((1, ED)),
            full((30, SD)), full((1, SD)), full((30, SD)), full((30, SD)),
            full((1, SD)), full((ED, ED)), full((ED, ED)),
        ],
        out_specs=(
            pl.BlockSpec((BN, ED), lambda i: (i, 0)),
            pl.BlockSpec((BN, ED), lambda i: (i, 0)),
            pl.BlockSpec((BN, ED), lambda i: (i, 0)),
            pl.BlockSpec((BN, SD), lambda i: (i, 0)),
            pl.BlockSpec((BN, SD), lambda i: (i, 0)),
            pl.BlockSpec((BN, SD), lambda i: (i, 0)),
            pl.BlockSpec((1, NG), lambda i: (0, 0)),
        ),
    )(posmax, inp30, batch2d, ew1t, eb1, ew2t, eb2, wsd, bsd, wsn, wsu,
      bsu, whd0t, whs0t)


def _edge_body(pre_ref, w2t_ref, b2_ref, m_ref):
    bc = BE // R
    sv = _silu(pre_ref[...].reshape(BE, ED))
    mv = _silu(jnp.dot(sv, w2t_ref[...],
                       preferred_element_type=jnp.float32) + b2_ref[...])
    m_ref[...] = mv.reshape(bc, R, ED)


def _edge_mlp(pre, w2t, b2):
    bc = BE // R
    return pl.pallas_call(
        _edge_body,
        grid=(GE,),
        out_shape=jax.ShapeDtypeStruct((NCHUNK, R, ED), jnp.float32),
        in_specs=[
            pl.BlockSpec((bc, R, ED), lambda i: (i, 0, 0)),
            pl.BlockSpec((ED, ED), lambda i: (0, 0)),
            pl.BlockSpec((1, ED), lambda i: (0, 0)),
        ],
        out_specs=pl.BlockSpec((bc, R, ED), lambda i: (i, 0, 0)),
    )(pre, w2t, b2)


def _upd_body(h_ref, p_ref, deg_ref, su_ref, uht_ref, uat_ref, u2t_ref,
              ub2_ref, batch_ref, hn_ref, gsum_ref, gsq_ref):
    psum = p_ref[0] + p_ref[1]
    degv = jnp.maximum(deg_ref[0, :, 0:1] + deg_ref[1, :, 0:1], 1.0)
    agg = psum * (1.0 / degv)
    u1 = _silu(jnp.dot(h_ref[...], uht_ref[...],
                       preferred_element_type=jnp.float32)
               + jnp.dot(agg, uat_ref[...],
                         preferred_element_type=jnp.float32)
               + su_ref[...])
    up = _silu(jnp.dot(u1, u2t_ref[...],
                       preferred_element_type=jnp.float32) + ub2_ref[...])
    hn = h_ref[...] + up
    hn_ref[...] = hn
    mask = (batch_ref[...] ==
            lax.broadcasted_iota(jnp.int32, (1, NG), 1)).astype(jnp.float32)
    gs = lax.dot_general(mask, hn, (((0,), (0,)), ((), ())),
                         preferred_element_type=jnp.float32)
    gq = lax.dot_general(mask, hn * hn, (((0,), (0,)), ((), ())),
                         preferred_element_type=jnp.float32)

    @pl.when(pl.program_id(0) == 0)
    def _():
        gsum_ref[...] = jnp.zeros_like(gsum_ref)
        gsq_ref[...] = jnp.zeros_like(gsq_ref)

    gsum_ref[...] += gs
    gsq_ref[...] += gq


def _update(h, partials, deg2, su_all, layer, uht, uat, u2t, ub2, batch2d):
    return pl.pallas_call(
        _upd_body,
        grid=(GN,),
        out_shape=(
            jax.ShapeDtypeStruct((N, ED), jnp.float32),
            jax.ShapeDtypeStruct((NG, ED), jnp.float32),
            jax.ShapeDtypeStruct((NG, ED), jnp.float32),
        ),
        in_specs=[
            pl.BlockSpec((BN, ED), lambda i: (i, 0)),
            pl.BlockSpec((NC, BN, ED), lambda i: (0, i, 0)),
            pl.BlockSpec((NC, BN, ED), lambda i: (0, i, 0)),
            pl.BlockSpec((BN, ED), lambda i, L=layer: (i, L)),
            pl.BlockSpec((ED, ED), lambda i: (0, 0)),
            pl.BlockSpec((ED, ED), lambda i: (0, 0)),
            pl.BlockSpec((ED, ED), lambda i: (0, 0)),
            pl.BlockSpec((1, ED), lambda i: (0, 0)),
            pl.BlockSpec((BN, 1), lambda i: (i, 0)),
        ],
        out_specs=(
            pl.BlockSpec((BN, ED), lambda i: (i, 0)),
            pl.BlockSpec((NG, ED), lambda i: (0, 0)),
            pl.BlockSpec((NG, ED), lambda i: (0, 0)),
        ),
    )(h, partials, deg2, su_all, uht, uat, u2t, ub2, batch2d)


def _norm_body_proj(hn_ref, gsum_ref, gsq_ref, cb_ref, batch_ref, whdt_ref,
                    whst_ref, sd_ref, ssn_ref, h_ref, a_ref, b_ref):
    hb = _norm_common(hn_ref, gsum_ref, gsq_ref, cb_ref, batch_ref)
    h_ref[...] = hb
    a_ref[...] = jnp.dot(hb, whdt_ref[...],
                         preferred_element_type=jnp.float32) + sd_ref[...]
    b_ref[...] = jnp.dot(hb, whst_ref[...],
                         preferred_element_type=jnp.float32) + ssn_ref[...]


def _norm_body_last(hn_ref, gsum_ref, gsq_ref, cb_ref, batch_ref, h_ref):
    h_ref[...] = _norm_common(hn_ref, gsum_ref, gsq_ref, cb_ref, batch_ref)


def _norm_common(hn_ref, gsum_ref, gsq_ref, cb_ref, batch_ref):
    cbm = jnp.maximum(cb_ref[...], 1.0)
    mask = (batch_ref[...] ==
            lax.broadcasted_iota(jnp.int32, (1, NG), 1)).astype(jnp.float32)
    maskc = mask * (1.0 / cbm)
    meanr = jnp.dot(maskc, gsum_ref[...], preferred_element_type=jnp.float32)
    eh2r = jnp.dot(maskc, gsq_ref[...], preferred_element_type=jnp.float32)
    varr = jnp.maximum(eh2r - meanr * meanr, 0.0)
    return (hn_ref[...] - meanr) * lax.rsqrt(varr + EPS)


def _norm_proj(hn, gsum, gsq, cb, batch2d, whdt, whst, sd_all, ssn_all,
               layer_next):
    return pl.pallas_call(
        _norm_body_proj,
        grid=(GN,),
        out_shape=(
            jax.ShapeDtypeStruct((N, ED), jnp.float32),
            jax.ShapeDtypeStruct((N, ED), jnp.float32),
            jax.ShapeDtypeStruct((N, ED), jnp.float32),
        ),
        in_specs=[
            pl.BlockSpec((BN, ED), lambda i: (i, 0)),
            pl.BlockSpec((NG, ED), lambda i: (0, 0)),
            pl.BlockSpec((NG, ED), lambda i: (0, 0)),
            pl.BlockSpec((1, NG), lambda i: (0, 0)),
            pl.BlockSpec((BN, 1), lambda i: (i, 0)),
            pl.BlockSpec((ED, ED), lambda i: (0, 0)),
            pl.BlockSpec((ED, ED), lambda i: (0, 0)),
            pl.BlockSpec((BN, ED), lambda i, L=layer_next: (i, L)),
            pl.BlockSpec((BN, ED), lambda i, L=layer_next: (i, L)),
        ],
        out_specs=(
            pl.BlockSpec((BN, ED), lambda i: (i, 0)),
            pl.BlockSpec((BN, ED), lambda i: (i, 0)),
            pl.BlockSpec((BN, ED), lambda i: (i, 0)),
        ),
    )(hn, gsum, gsq, cb, batch2d, whdt, whst, sd_all, ssn_all)


def _norm_last(hn, gsum, gsq, cb, batch2d):
    return pl.pallas_call(
        _norm_body_last,
        grid=(GN,),
        out_shape=jax.ShapeDtypeStruct((N, ED), jnp.float32),
        in_specs=[
            pl.BlockSpec((BN, ED), lambda i: (i, 0)),
            pl.BlockSpec((NG, ED), lambda i: (0, 0)),
            pl.BlockSpec((NG, ED), lambda i: (0, 0)),
            pl.BlockSpec((1, NG), lambda i: (0, 0)),
            pl.BlockSpec((BN, 1), lambda i: (i, 0)),
        ],
        out_specs=pl.BlockSpec((BN, ED), lambda i: (i, 0)),
    )(hn, gsum, gsq, cb, batch2d)


def _dec_body(h_ref, x_ref, dt_ref, w1d_ref, b1d_ref, w2d_ref, b2s_ref,
              out_ref):
    z1 = _silu(jnp.dot(h_ref[...], w1d_ref[...],
                       preferred_element_type=jnp.float32) + b1d_ref[...])
    z2 = jnp.dot(z1, w2d_ref[...],
                 preferred_element_type=jnp.float32) + b2s_ref[0, 0]
    steps = (lax.broadcasted_iota(jnp.int32, (1, TW), 1) + 1
             ).astype(jnp.float32)
    dtv = dt_ref[0, 0] * steps
    out_ref[...] = x_ref[:, TW - 1:TW] + dtv * z2


def _decoder(h, x, dt2d, w1d, b1d, w2d, b2s):
    return pl.pallas_call(
        _dec_body,
        grid=(GN,),
        out_shape=jax.ShapeDtypeStruct((N, TW), jnp.float32),
        in_specs=[
            pl.BlockSpec((BN, ED), lambda i: (i, 0)),
            pl.BlockSpec((BN, TW), lambda i: (i, 0)),
            pl.BlockSpec(memory_space=pltpu.SMEM),
            pl.BlockSpec((ED, 8 * 38), lambda i: (0, 0)),
            pl.BlockSpec((1, 8 * 38), lambda i: (0, 0)),
            pl.BlockSpec((8 * 38, TW), lambda i: (0, 0)),
            pl.BlockSpec(memory_space=pltpu.SMEM),
        ],
        out_specs=pl.BlockSpec((BN, TW), lambda i: (i, 0)),
    )(h, x, dt2d, w1d, b1d, w2d, b2s)


def _densify_convs(conv1_w, conv1_b, conv2_w, conv2_b):
    o_i, p_i, k_i = np.meshgrid(np.arange(8), np.arange(38), np.arange(16),
                                indexing="ij")
    rows1 = (3 * p_i + k_i).reshape(-1)
    cols1 = (o_i * 38 + p_i).reshape(-1)
    w1d = jnp.zeros((ED, 8 * 38), jnp.float32).at[rows1, cols1].set(
        conv1_w[o_i.reshape(-1), 0, k_i.reshape(-1)])
    b1d = jnp.repeat(conv1_b, 38).reshape(1, 8 * 38)
    o_i, q_i, d_i = np.meshgrid(np.arange(8), np.arange(TW), np.arange(14),
                                indexing="ij")
    rows2 = (o_i * 38 + q_i + d_i).reshape(-1)
    cols2 = q_i.reshape(-1)
    w2d = jnp.zeros((8 * 38, TW), jnp.float32).at[rows2, cols2].set(
        conv2_w[0, o_i.reshape(-1), d_i.reshape(-1)])
    b2s = conv2_b.reshape(1, 1)
    return w1d, b1d, w2d, b2s


def kernel(x, pos, t, vars_abc, dt, enc_w1, enc_b1, enc_w2, enc_b2,
           msg1_w, msg1_b, msg2_w, msg2_b, upd1_w, upd1_b, upd2_w, upd2_b,
           conv1_w, conv1_b, conv2_w, conv2_b, edge_index, batch):
    f32 = jnp.float32
    SD = NL * ED

    inp30 = jnp.concatenate([x, pos, t, vars_abc], axis=-1)
    batch2d = batch.reshape(N, 1).astype(jnp.int32)
    src3d = edge_index[0].astype(jnp.int32).reshape(NW, J, R)
    dst3d = edge_index[1].astype(jnp.int32).reshape(NW, J, R)
    pos_pad = jnp.concatenate(
        [pos.reshape(-1), jnp.zeros((240,), f32)]).reshape(80, 128)

    whd_t = jnp.transpose(msg1_w[:, :, :ED], (0, 2, 1))
    whs_t = jnp.transpose(msg1_w[:, :, ED:2 * ED], (0, 2, 1))
    wx = msg1_w[:, :, 2 * ED:2 * ED + TW]
    wp = msg1_w[:, :, 2 * ED + TW:2 * ED + TW + 1]
    wv = msg1_w[:, :, 2 * ED + TW + 1:]
    wsd = jnp.concatenate([wx, wp, wv], axis=2)
    wsd_t = wsd.reshape(SD, 30).T
    bsd = msg1_b.reshape(1, SD)
    wsn = -jnp.concatenate([wx, wp, jnp.zeros((NL, ED, 4), f32)], axis=2)
    wsn_t = wsn.reshape(SD, 30).T
    uv = upd1_w[:, :, 2 * ED:]
    wsu = jnp.concatenate([jnp.zeros((NL, ED, 26), f32), uv], axis=2)
    wsu_t = wsu.reshape(SD, 30).T
    bsu = upd1_b.reshape(1, SD)
    uh_t = jnp.transpose(upd1_w[:, :, :ED], (0, 2, 1))
    ua_t = jnp.transpose(upd1_w[:, :, ED:2 * ED], (0, 2, 1))
    w2_t = jnp.transpose(msg2_w, (0, 2, 1))
    u2_t = jnp.transpose(upd2_w, (0, 2, 1))
    ew1t = enc_w1.T
    ew2t = enc_w2.T
    eb1 = enc_b1.reshape(1, ED)
    eb2 = enc_b2.reshape(1, ED)
    mb2 = msg2_b.reshape(NL, 1, ED)
    ub2 = upd2_b.reshape(NL, 1, ED)
    w1d, b1d, w2d, b2s = _densify_convs(conv1_w, conv1_b, conv2_w, conv2_b)
    dt2d = dt.reshape(1, 1)

    posmax = _posmax(pos_pad)
    h, a, b, sd_all, ssn_all, su_all, cb = _encoder(
        posmax, inp30, batch2d, ew1t, eb1, ew2t, eb2, wsd_t, bsd, wsn_t,
        wsu_t, bsu, whd_t[0].copy(), whs_t[0].copy())
    deg2 = _sc_deg(dst3d)

    for l in range(NL):
        pre = _sc_gather(a, b, dst3d, src3d)
        m = _edge_mlp(pre, w2_t[l], mb2[l])
        partials = _sc_scatter(m, dst3d)
        hn, gsum, gsq = _update(h, partials, deg2, su_all, l, uh_t[l],
                                ua_t[l], u2_t[l], ub2[l], batch2d)
        if l < NL - 1:
            h, a, b = _norm_proj(hn, gsum, gsq, cb, batch2d, whd_t[l + 1],
                                 whs_t[l + 1], sd_all, ssn_all, l + 1)
        else:
            h = _norm_last(hn, gsum, gsq, cb, batch2d)

    return _decoder(h, x, dt2d, w1d, b1d, w2d, b2s)


# speedup vs baseline: 7.7344x; 1.0044x over previous
import functools

import numpy as np
import jax
import jax.numpy as jnp
from jax import lax
from jax.experimental import pallas as pl
from jax.experimental.pallas import tpu as pltpu
from jax.experimental.pallas import tpu_sc as plsc

N = 10000
E = 320000
ED = 128
NL = 6
NG = 16
TW = 25
T_MAX = 4.0
EPS = 1e-5

NC = 2
NS = 16
NW = NC * NS
EPW = E // NW
R = 80
J = EPW // R
NCHUNK = E // R
NPAD = 10240
RPT = NPAD // NS

BN = 1000
GN = N // BN
BE = 2000
GE = E // BE

_SC_MESH = plsc.VectorSubcoreMesh(core_axis_name="c", subcore_axis_name="s")


def _silu(v):
    return v * jax.nn.sigmoid(v)


def _add3_rows(out_v, a_v, b_v):
    def row(r, carry):
        for jj in range(ED // 16):
            sl = pl.ds(jj * 16, 16)
            out_v[r, sl] = a_v[r, sl] + b_v[r, sl]
        return carry

    lax.fori_loop(0, R, row, 0)


def _sc_gather_common(a_hbm, b_hbm, dst_hbm, src_hbm, out_hbm,
                      idxd_v, idxs_v, a0, b0, a1, b1, o0, o1,
                      sa0, sb0, sa1, sb1, sw0, sw1, with_deg,
                      deg_hbm=None, ones_v=None, acc_sh=None, sd=None):
    c = lax.axis_index("c")
    s = lax.axis_index("s")
    wid = s * NC + c
    pltpu.sync_copy(dst_hbm.at[wid], idxd_v)
    pltpu.sync_copy(src_hbm.at[wid], idxs_v)
    base = wid * J

    if with_deg:
        def frow(r, carry):
            for jj in range(ED // 16):
                sl = pl.ds(jj * 16, 16)
                a0[r, sl] = jnp.zeros((16,), jnp.float32)
                ones_v[r, sl] = jnp.ones((16,), jnp.float32)
            return carry

        lax.fori_loop(0, R, frow, 0)
        for q in range(RPT // R):
            pltpu.sync_copy(a0, acc_sh.at[pl.ds(s * RPT + q * R, R)])
        plsc.subcore_barrier()

    def fire(ck_local, ra, rb, sa, sb):
        pltpu.async_copy(a_hbm.at[idxd_v.at[ck_local]], ra, sa)
        pltpu.async_copy(b_hbm.at[idxs_v.at[ck_local]], rb, sb)
        if with_deg:
            pltpu.async_copy(ones_v, acc_sh.at[idxd_v.at[ck_local]], sd,
                             add=True)

    def drain(ck_local, first, ra, rb, ro, sa, sb, sw):
        pltpu.make_async_copy(a_hbm.at[idxd_v.at[ck_local]], ra, sa).wait()
        pltpu.make_async_copy(b_hbm.at[idxs_v.at[ck_local]], rb, sb).wait()

        @pl.when(jnp.logical_not(first))
        def _():
            pltpu.make_async_copy(ro, out_hbm.at[base + ck_local],
                                  sw).wait()

        _add3_rows(ro, ra, rb)
        pltpu.async_copy(ro, out_hbm.at[base + ck_local], sw)

    fire(0, a0, b0, sa0, sb0)

    def step(i, carry):
        fire(2 * i + 1, a1, b1, sa1, sb1)
        drain(2 * i, i == 0, a0, b0, o0, sa0, sb0, sw0)
        fire(2 * i + 2, a0, b0, sa0, sb0)
        drain(2 * i + 1, i == 0, a1, b1, o1, sa1, sb1, sw1)
        return carry

    lax.fori_loop(0, (J - 1) // 2, step, 0, unroll=False)
    drain(J - 1, False, a0, b0, o0, sa0, sb0, sw0)
    pltpu.make_async_copy(o0, out_hbm.at[base + J - 1], sw0).wait()
    pltpu.make_async_copy(o1, out_hbm.at[base + J - 2], sw1).wait()
    if with_deg:
        def dstep(j, carry):
            pltpu.make_async_copy(ones_v, acc_sh.at[idxd_v.at[j]],
                                  sd).wait()
            return carry

        lax.fori_loop(0, J, dstep, 0)
        plsc.subcore_barrier()
        pltpu.sync_copy(acc_sh.at[pl.ds(s * RPT, RPT)],
                        deg_hbm.at[c, pl.ds(s * RPT, RPT)])


_GATHER_SCRATCH = [
    pltpu.VMEM((J, R), jnp.int32),
    pltpu.VMEM((J, R), jnp.int32),
    pltpu.VMEM((R, ED), jnp.float32),
    pltpu.VMEM((R, ED), jnp.float32),
    pltpu.VMEM((R, ED), jnp.float32),
    pltpu.VMEM((R, ED), jnp.float32),
    pltpu.VMEM((R, ED), jnp.float32),
    pltpu.VMEM((R, ED), jnp.float32),
    pltpu.SemaphoreType.DMA,
    pltpu.SemaphoreType.DMA,
    pltpu.SemaphoreType.DMA,
    pltpu.SemaphoreType.DMA,
    pltpu.SemaphoreType.DMA,
    pltpu.SemaphoreType.DMA,
]


def _sc_gather(a, b, dst3d, src3d):
    def body(a_hbm, b_hbm, dst_hbm, src_hbm, out_hbm,
             idxd_v, idxs_v, a0, b0, a1, b1, o0, o1,
             sa0, sb0, sa1, sb1, sw0, sw1):
        _sc_gather_common(a_hbm, b_hbm, dst_hbm, src_hbm, out_hbm,
                          idxd_v, idxs_v, a0, b0, a1, b1, o0, o1,
                          sa0, sb0, sa1, sb1, sw0, sw1, False)

    f = pl.kernel(
        body,
        out_type=jax.ShapeDtypeStruct((NCHUNK, R, ED), jnp.float32),
        mesh=_SC_MESH,
        scratch_types=list(_GATHER_SCRATCH),
    )
    return f(a, b, dst3d, src3d)


def _sc_gather_deg(a, b, dst3d, src3d):
    def body(a_hbm, b_hbm, dst_hbm, src_hbm, out_hbm, deg_hbm,
             idxd_v, idxs_v, a0, b0, a1, b1, o0, o1,
             sa0, sb0, sa1, sb1, sw0, sw1, ones_v, acc_sh, sd):
        _sc_gather_common(a_hbm, b_hbm, dst_hbm, src_hbm, out_hbm,
                          idxd_v, idxs_v, a0, b0, a1, b1, o0, o1,
                          sa0, sb0, sa1, sb1, sw0, sw1, True,
                          deg_hbm=deg_hbm, ones_v=ones_v, acc_sh=acc_sh,
                          sd=sd)

    f = pl.kernel(
        body,
        out_type=(jax.ShapeDtypeStruct((NCHUNK, R, ED), jnp.float32),
                  jax.ShapeDtypeStruct((NC, NPAD, ED), jnp.float32)),
        mesh=_SC_MESH,
        scratch_types=list(_GATHER_SCRATCH) + [
            pltpu.VMEM((R, ED), jnp.float32),
            pltpu.VMEM_SHARED((NPAD, ED), jnp.float32),
            pltpu.SemaphoreType.DMA,
        ],
    )
    return f(a, b, dst3d, src3d)


def _sc_scatter_body(m_hbm, dst_hbm, out_hbm, idx_v, m0_v, m1_v, acc_sh,
                     s0, s1, ss0, ss1):
    c = lax.axis_index("c")
    s = lax.axis_index("s")
    wid = s * NC + c

    def zrow(r, carry):
        for jj in range(ED // 16):
            m0_v[r, pl.ds(jj * 16, 16)] = jnp.zeros((16,), jnp.float32)
        return carry

    lax.fori_loop(0, R, zrow, 0)
    for q in range(RPT // R):
        pltpu.sync_copy(m0_v, acc_sh.at[pl.ds(s * RPT + q * R, R)])
    pltpu.sync_copy(dst_hbm.at[wid], idx_v)
    plsc.subcore_barrier()
    base = wid * J

    pltpu.async_copy(m_hbm.at[base], m0_v, s0)

    def step(i, carry):
        @pl.when(i > 0)
        def _():
            pltpu.make_async_copy(m1_v, acc_sh.at[idx_v.at[2 * i - 1]],
                                  ss1).wait()

        pltpu.async_copy(m_hbm.at[base + 2 * i + 1], m1_v, s1)
        pltpu.make_async_copy(m_hbm.at[base + 2 * i], m0_v, s0).wait()
        pltpu.async_copy(m0_v, acc_sh.at[idx_v.at[2 * i]], ss0, add=True)
        pltpu.make_async_copy(m_hbm.at[base + 2 * i + 1], m1_v, s1).wait()
        pltpu.make_async_copy(m0_v, acc_sh.at[idx_v.at[2 * i]], ss0).wait()
        pltpu.async_copy(m_hbm.at[base + 2 * i + 2], m0_v, s0)
        pltpu.async_copy(m1_v, acc_sh.at[idx_v.at[2 * i + 1]], ss1,
                         add=True)
        return carry

    lax.fori_loop(0, (J - 1) // 2, step, 0)
    pltpu.make_async_copy(m_hbm.at[base + J - 1], m0_v, s0).wait()
    pltpu.async_copy(m0_v, acc_sh.at[idx_v.at[J - 1]], ss0, add=True)
    pltpu.make_async_copy(m0_v, acc_sh.at[idx_v.at[J - 1]], ss0).wait()
    pltpu.make_async_copy(m1_v, acc_sh.at[idx_v.at[J - 2]], ss1).wait()
    plsc.subcore_barrier()
    pltpu.sync_copy(acc_sh.at[pl.ds(s * RPT, RPT)],
                    out_hbm.at[c, pl.ds(s * RPT, RPT)])


def _sc_scatter(m, dst3d):
    f = pl.kernel(
        _sc_scatter_body,
        out_type=jax.ShapeDtypeStruct((NC, NPAD, ED), jnp.float32),
        mesh=_SC_MESH,
        scratch_types=[
            pltpu.VMEM((J, R), jnp.int32),
            pltpu.VMEM((R, ED), jnp.float32),
            pltpu.VMEM((R, ED), jnp.float32),
            pltpu.VMEM_SHARED((NPAD, ED), jnp.float32),
            pltpu.SemaphoreType.DMA,
            pltpu.SemaphoreType.DMA,
            pltpu.SemaphoreType.DMA,
            pltpu.SemaphoreType.DMA,
        ],
    )
    return f(m, dst3d)


def _sc_deg_body(dst_hbm, out_hbm, idx_v, ones_v, z_v, acc_sh):
    c = lax.axis_index("c")
    s = lax.axis_index("s")
    wid = s * NC + c

    def frow(r, carry):
        for jj in range(ED // 16):
            ones_v[r, pl.ds(jj * 16, 16)] = jnp.ones((16,), jnp.float32)
            z_v[r, pl.ds(jj * 16, 16)] = jnp.zeros((16,), jnp.float32)
        return carry

    lax.fori_loop(0, R, frow, 0)
    for q in range(RPT // R):
        pltpu.sync_copy(z_v, acc_sh.at[pl.ds(s * RPT + q * R, R)])
    pltpu.sync_copy(dst_hbm.at[wid], idx_v)
    plsc.subcore_barrier()

    def step(j, carry):
        pltpu.sync_copy(ones_v, acc_sh.at[idx_v.at[j]], add=True)
        return carry

    lax.fori_loop(0, J, step, 0)
    plsc.subcore_barrier()
    pltpu.sync_copy(acc_sh.at[pl.ds(s * RPT, RPT)],
                    out_hbm.at[c, pl.ds(s * RPT, RPT)])


def _sc_deg(dst3d):
    f = pl.kernel(
        _sc_deg_body,
        out_type=jax.ShapeDtypeStruct((NC, NPAD, ED), jnp.float32),
        mesh=_SC_MESH,
        scratch_types=[
            pltpu.VMEM((J, R), jnp.int32),
            pltpu.VMEM((R, ED), jnp.float32),
            pltpu.VMEM((R, ED), jnp.float32),
            pltpu.VMEM_SHARED((NPAD, ED), jnp.float32),
        ],
    )
    return f(dst3d)


def _posmax_body(pos_ref, out_ref):
    out_ref[0, 0] = jnp.max(pos_ref[...])


def _posmax(pos_pad):
    return pl.pallas_call(
        _posmax_body,
        out_shape=jax.ShapeDtypeStruct((1, 1), jnp.float32),
        in_specs=[pl.BlockSpec(pos_pad.shape, lambda: (0, 0))],
        out_specs=pl.BlockSpec((1, 1), lambda: (0, 0),
                               memory_space=pltpu.SMEM),
    )(pos_pad)


def _enc_body(posmax_ref, inp_ref, batch_ref, ew1t_ref, eb1_ref, ew2t_ref,
              eb2_ref, wsd_ref, bsd_ref, wsn_ref, wsu_ref, bsu_ref,
              whd0t_ref, whs0t_ref,
              h0_ref, a0_ref, b0_ref, sd_ref, ssn_ref, su_ref, cb_ref):
    col = lax.broadcasted_iota(jnp.int32, (1, 30), 1)
    inv_pm = 1.0 / posmax_ref[0, 0]
    scale = jnp.where(col == 25, inv_pm,
                      jnp.where(col == 26, 1.0 / T_MAX, 1.0))
    xb = inp_ref[...] * scale
    h1 = _silu(jnp.dot(xb, ew1t_ref[...],
                       preferred_element_type=jnp.float32) + eb1_ref[...])
    h0 = _silu(jnp.dot(h1, ew2t_ref[...],
                       preferred_element_type=jnp.float32) + eb2_ref[...])
    sd = jnp.dot(xb, wsd_ref[...],
                 preferred_element_type=jnp.float32) + bsd_ref[...]
    ssn = jnp.dot(xb, wsn_ref[...], preferred_element_type=jnp.float32)
    su = jnp.dot(xb, wsu_ref[...],
                 preferred_element_type=jnp.float32) + bsu_ref[...]
    h0_ref[...] = h0
    a0_ref[...] = jnp.dot(h0, whd0t_ref[...],
                          preferred_element_type=jnp.float32) + sd[:, :ED]
    b0_ref[...] = jnp.dot(h0, whs0t_ref[...],
                          preferred_element_type=jnp.float32) + ssn[:, :ED]
    sd_ref[...] = sd
    ssn_ref[...] = ssn
    su_ref[...] = su
    mask = (batch_ref[...] ==
            lax.broadcasted_iota(jnp.int32, (1, NG), 1)).astype(jnp.float32)
    part = jnp.sum(mask, axis=0, keepdims=True)

    @pl.when(pl.program_id(0) == 0)
    def _():
        cb_ref[...] = jnp.zeros_like(cb_ref)

    cb_ref[...] += part


def _encoder(posmax, inp30, batch2d, ew1t, eb1, ew2t, eb2, wsd, bsd, wsn,
             wsu, bsu, whd0t, whs0t):
    SD = NL * ED
    full = lambda shp: pl.BlockSpec(shp, lambda i: (0, 0))
    return pl.pallas_call(
        _enc_body,
        grid=(GN,),
        out_shape=(
            jax.ShapeDtypeStruct((N, ED), jnp.float32),
            jax.ShapeDtypeStruct((N, ED), jnp.float32),
            jax.ShapeDtypeStruct((N, ED), jnp.float32),
            jax.ShapeDtypeStruct((N, SD), jnp.float32),
            jax.ShapeDtypeStruct((N, SD), jnp.float32),
            jax.ShapeDtypeStruct((N, SD), jnp.float32),
            jax.ShapeDtypeStruct((1, NG), jnp.float32),
        ),
        in_specs=[
            pl.BlockSpec(memory_space=pltpu.SMEM),
            pl.BlockSpec((BN, 30), lambda i: (i, 0)),
            pl.BlockSpec((BN, 1), lambda i: (i, 0)),
            full((30, ED)), full((1, ED)), full((ED, ED)), full((1, ED)),
            full((30, SD)), full((1, SD)), full((30, SD)), full((30, SD)),
            full((1, SD)), full((ED, ED)), full((ED, ED)),
        ],
        out_specs=(
            pl.BlockSpec((BN, ED), lambda i: (i, 0)),
            pl.BlockSpec((BN, ED), lambda i: (i, 0)),
            pl.BlockSpec((BN, ED), lambda i: (i, 0)),
            pl.BlockSpec((BN, SD), lambda i: (i, 0)),
            pl.BlockSpec((BN, SD), lambda i: (i, 0)),
            pl.BlockSpec((BN, SD), lambda i: (i, 0)),
            pl.BlockSpec((1, NG), lambda i: (0, 0)),
        ),
    )(posmax, inp30, batch2d, ew1t, eb1, ew2t, eb2, wsd, bsd, wsn, wsu,
      bsu, whd0t, whs0t)


def _edge_body(pre_ref, w2t_ref, b2_ref, m_ref):
    bc = BE // R
    sv = _silu(pre_ref[...].reshape(BE, ED))
    mv = _silu(jnp.dot(sv.astype(jnp.bfloat16),
                       w2t_ref[...].astype(jnp.bfloat16),
                       preferred_element_type=jnp.float32) + b2_ref[...])
    m_ref[...] = mv.reshape(bc, R, ED)


def _edge_mlp(pre, w2t, b2):
    bc = BE // R
    return pl.pallas_call(
        _edge_body,
        grid=(GE,),
        out_shape=jax.ShapeDtypeStruct((NCHUNK, R, ED), jnp.float32),
        in_specs=[
            pl.BlockSpec((bc, R, ED), lambda i: (i, 0, 0)),
            pl.BlockSpec((ED, ED), lambda i: (0, 0)),
            pl.BlockSpec((1, ED), lambda i: (0, 0)),
        ],
        out_specs=pl.BlockSpec((bc, R, ED), lambda i: (i, 0, 0)),
    )(pre, w2t, b2)


def _upd_body(h_ref, p_ref, deg_ref, su_ref, uht_ref, uat_ref, u2t_ref,
              ub2_ref, batch_ref, hn_ref, gsum_ref, gsq_ref):
    psum = p_ref[0] + p_ref[1]
    degv = jnp.maximum(deg_ref[0, :, 0:1] + deg_ref[1, :, 0:1], 1.0)
    agg = psum * (1.0 / degv)
    bf = jnp.bfloat16
    u1 = _silu(jnp.dot(h_ref[...].astype(bf), uht_ref[...].astype(bf),
                       preferred_element_type=jnp.float32)
               + jnp.dot(agg.astype(bf), uat_ref[...].astype(bf),
                         preferred_element_type=jnp.float32)
               + su_ref[...])
    up = _silu(jnp.dot(u1.astype(bf), u2t_ref[...].astype(bf),
                       preferred_element_type=jnp.float32) + ub2_ref[...])
    hn = h_ref[...] + up
    hn_ref[...] = hn
    mask = (batch_ref[...] ==
            lax.broadcasted_iota(jnp.int32, (1, NG), 1)).astype(jnp.float32)
    gs = lax.dot_general(mask, hn, (((0,), (0,)), ((), ())),
                         preferred_element_type=jnp.float32)
    gq = lax.dot_general(mask, hn * hn, (((0,), (0,)), ((), ())),
                         preferred_element_type=jnp.float32)

    @pl.when(pl.program_id(0) == 0)
    def _():
        gsum_ref[...] = jnp.zeros_like(gsum_ref)
        gsq_ref[...] = jnp.zeros_like(gsq_ref)

    gsum_ref[...] += gs
    gsq_ref[...] += gq


def _update(h, partials, deg2, su_all, layer, uht, uat, u2t, ub2, batch2d):
    return pl.pallas_call(
        _upd_body,
        grid=(GN,),
        out_shape=(
            jax.ShapeDtypeStruct((N, ED), jnp.float32),
            jax.ShapeDtypeStruct((NG, ED), jnp.float32),
            jax.ShapeDtypeStruct((NG, ED), jnp.float32),
        ),
        in_specs=[
            pl.BlockSpec((BN, ED), lambda i: (i, 0)),
            pl.BlockSpec((NC, BN, ED), lambda i: (0, i, 0)),
            pl.BlockSpec((NC, BN, ED), lambda i: (0, i, 0)),
            pl.BlockSpec((BN, ED), lambda i, L=layer: (i, L)),
            pl.BlockSpec((ED, ED), lambda i: (0, 0)),
            pl.BlockSpec((ED, ED), lambda i: (0, 0)),
            pl.BlockSpec((ED, ED), lambda i: (0, 0)),
            pl.BlockSpec((1, ED), lambda i: (0, 0)),
            pl.BlockSpec((BN, 1), lambda i: (i, 0)),
        ],
        out_specs=(
            pl.BlockSpec((BN, ED), lambda i: (i, 0)),
            pl.BlockSpec((NG, ED), lambda i: (0, 0)),
            pl.BlockSpec((NG, ED), lambda i: (0, 0)),
        ),
    )(h, partials, deg2, su_all, uht, uat, u2t, ub2, batch2d)


def _norm_body_proj(hn_ref, gsum_ref, gsq_ref, cb_ref, batch_ref, whdt_ref,
                    whst_ref, sd_ref, ssn_ref, h_ref, a_ref, b_ref):
    hb = _norm_common(hn_ref, gsum_ref, gsq_ref, cb_ref, batch_ref)
    h_ref[...] = hb
    a_ref[...] = jnp.dot(hb, whdt_ref[...],
                         preferred_element_type=jnp.float32) + sd_ref[...]
    b_ref[...] = jnp.dot(hb, whst_ref[...],
                         preferred_element_type=jnp.float32) + ssn_ref[...]


def _norm_body_last(hn_ref, gsum_ref, gsq_ref, cb_ref, batch_ref, h_ref):
    h_ref[...] = _norm_common(hn_ref, gsum_ref, gsq_ref, cb_ref, batch_ref)


def _norm_common(hn_ref, gsum_ref, gsq_ref, cb_ref, batch_ref):
    cbm = jnp.maximum(cb_ref[...], 1.0)
    mask = (batch_ref[...] ==
            lax.broadcasted_iota(jnp.int32, (1, NG), 1)).astype(jnp.float32)
    maskc = mask * (1.0 / cbm)
    meanr = jnp.dot(maskc, gsum_ref[...], preferred_element_type=jnp.float32)
    eh2r = jnp.dot(maskc, gsq_ref[...], preferred_element_type=jnp.float32)
    varr = jnp.maximum(eh2r - meanr * meanr, 0.0)
    return (hn_ref[...] - meanr) * lax.rsqrt(varr + EPS)


def _norm_proj(hn, gsum, gsq, cb, batch2d, whdt, whst, sd_all, ssn_all,
               layer_next):
    return pl.pallas_call(
        _norm_body_proj,
        grid=(GN,),
        out_shape=(
            jax.ShapeDtypeStruct((N, ED), jnp.float32),
            jax.ShapeDtypeStruct((N, ED), jnp.float32),
            jax.ShapeDtypeStruct((N, ED), jnp.float32),
        ),
        in_specs=[
            pl.BlockSpec((BN, ED), lambda i: (i, 0)),
            pl.BlockSpec((NG, ED), lambda i: (0, 0)),
            pl.BlockSpec((NG, ED), lambda i: (0, 0)),
            pl.BlockSpec((1, NG), lambda i: (0, 0)),
            pl.BlockSpec((BN, 1), lambda i: (i, 0)),
            pl.BlockSpec((ED, ED), lambda i: (0, 0)),
            pl.BlockSpec((ED, ED), lambda i: (0, 0)),
            pl.BlockSpec((BN, ED), lambda i, L=layer_next: (i, L)),
            pl.BlockSpec((BN, ED), lambda i, L=layer_next: (i, L)),
        ],
        out_specs=(
            pl.BlockSpec((BN, ED), lambda i: (i, 0)),
            pl.BlockSpec((BN, ED), lambda i: (i, 0)),
            pl.BlockSpec((BN, ED), lambda i: (i, 0)),
        ),
    )(hn, gsum, gsq, cb, batch2d, whdt, whst, sd_all, ssn_all)


def _norm_last(hn, gsum, gsq, cb, batch2d):
    return pl.pallas_call(
        _norm_body_last,
        grid=(GN,),
        out_shape=jax.ShapeDtypeStruct((N, ED), jnp.float32),
        in_specs=[
            pl.BlockSpec((BN, ED), lambda i: (i, 0)),
            pl.BlockSpec((NG, ED), lambda i: (0, 0)),
            pl.BlockSpec((NG, ED), lambda i: (0, 0)),
            pl.BlockSpec((1, NG), lambda i: (0, 0)),
            pl.BlockSpec((BN, 1), lambda i: (i, 0)),
        ],
        out_specs=pl.BlockSpec((BN, ED), lambda i: (i, 0)),
    )(hn, gsum, gsq, cb, batch2d)


def _dec_body(h_ref, x_ref, dt_ref, w1d_ref, b1d_ref, w2d_ref, b2s_ref,
              out_ref):
    z1 = _silu(jnp.dot(h_ref[...], w1d_ref[...],
                       preferred_element_type=jnp.float32) + b1d_ref[...])
    z2 = jnp.dot(z1, w2d_ref[...],
                 preferred_element_type=jnp.float32) + b2s_ref[0, 0]
    steps = (lax.broadcasted_iota(jnp.int32, (1, TW), 1) + 1
             ).astype(jnp.float32)
    dtv = dt_ref[0, 0] * steps
    out_ref[...] = x_ref[:, TW - 1:TW] + dtv * z2


def _decoder(h, x, dt2d, w1d, b1d, w2d, b2s):
    return pl.pallas_call(
        _dec_body,
        grid=(GN,),
        out_shape=jax.ShapeDtypeStruct((N, TW), jnp.float32),
        in_specs=[
            pl.BlockSpec((BN, ED), lambda i: (i, 0)),
            pl.BlockSpec((BN, TW), lambda i: (i, 0)),
            pl.BlockSpec(memory_space=pltpu.SMEM),
            pl.BlockSpec((ED, 8 * 38), lambda i: (0, 0)),
            pl.BlockSpec((1, 8 * 38), lambda i: (0, 0)),
            pl.BlockSpec((8 * 38, TW), lambda i: (0, 0)),
            pl.BlockSpec(memory_space=pltpu.SMEM),
        ],
        out_specs=pl.BlockSpec((BN, TW), lambda i: (i, 0)),
    )(h, x, dt2d, w1d, b1d, w2d, b2s)


def _densify_convs(conv1_w, conv1_b, conv2_w, conv2_b):
    o_i, p_i, k_i = np.meshgrid(np.arange(8), np.arange(38), np.arange(16),
                                indexing="ij")
    rows1 = (3 * p_i + k_i).reshape(-1)
    cols1 = (o_i * 38 + p_i).reshape(-1)
    w1d = jnp.zeros((ED, 8 * 38), jnp.float32).at[rows1, cols1].set(
        conv1_w[o_i.reshape(-1), 0, k_i.reshape(-1)])
    b1d = jnp.repeat(conv1_b, 38).reshape(1, 8 * 38)
    o_i, q_i, d_i = np.meshgrid(np.arange(8), np.arange(TW), np.arange(14),
                                indexing="ij")
    rows2 = (o_i * 38 + q_i + d_i).reshape(-1)
    cols2 = q_i.reshape(-1)
    w2d = jnp.zeros((8 * 38, TW), jnp.float32).at[rows2, cols2].set(
        conv2_w[0, o_i.reshape(-1), d_i.reshape(-1)])
    b2s = conv2_b.reshape(1, 1)
    return w1d, b1d, w2d, b2s


def kernel(x, pos, t, vars_abc, dt, enc_w1, enc_b1, enc_w2, enc_b2,
           msg1_w, msg1_b, msg2_w, msg2_b, upd1_w, upd1_b, upd2_w, upd2_b,
           conv1_w, conv1_b, conv2_w, conv2_b, edge_index, batch):
    f32 = jnp.float32
    SD = NL * ED

    inp30 = jnp.concatenate([x, pos, t, vars_abc], axis=-1)
    batch2d = batch.reshape(N, 1).astype(jnp.int32)
    src3d = edge_index[0].astype(jnp.int32).reshape(NW, J, R)
    dst3d = edge_index[1].astype(jnp.int32).reshape(NW, J, R)
    pos_pad = jnp.concatenate(
        [pos.reshape(-1), jnp.zeros((240,), f32)]).reshape(80, 128)

    whd_t = jnp.transpose(msg1_w[:, :, :ED], (0, 2, 1))
    whs_t = jnp.transpose(msg1_w[:, :, ED:2 * ED], (0, 2, 1))
    wx = msg1_w[:, :, 2 * ED:2 * ED + TW]
    wp = msg1_w[:, :, 2 * ED + TW:2 * ED + TW + 1]
    wv = msg1_w[:, :, 2 * ED + TW + 1:]
    wsd = jnp.concatenate([wx, wp, wv], axis=2)
    wsd_t = wsd.reshape(SD, 30).T
    bsd = msg1_b.reshape(1, SD)
    wsn = -jnp.concatenate([wx, wp, jnp.zeros((NL, ED, 4), f32)], axis=2)
    wsn_t = wsn.reshape(SD, 30).T
    uv = upd1_w[:, :, 2 * ED:]
    wsu = jnp.concatenate([jnp.zeros((NL, ED, 26), f32), uv], axis=2)
    wsu_t = wsu.reshape(SD, 30).T
    bsu = upd1_b.reshape(1, SD)
    uh_t = jnp.transpose(upd1_w[:, :, :ED], (0, 2, 1))
    ua_t = jnp.transpose(upd1_w[:, :, ED:2 * ED], (0, 2, 1))
    w2_t = jnp.transpose(msg2_w, (0, 2, 1))
    u2_t = jnp.transpose(upd2_w, (0, 2, 1))
    ew1t = enc_w1.T
    ew2t = enc_w2.T
    eb1 = enc_b1.reshape(1, ED)
    eb2 = enc_b2.reshape(1, ED)
    mb2 = msg2_b.reshape(NL, 1, ED)
    ub2 = upd2_b.reshape(NL, 1, ED)
    w1d, b1d, w2d, b2s = _densify_convs(conv1_w, conv1_b, conv2_w, conv2_b)
    dt2d = dt.reshape(1, 1)

    posmax = _posmax(pos_pad)
    h, a, b, sd_all, ssn_all, su_all, cb = _encoder(
        posmax, inp30, batch2d, ew1t, eb1, ew2t, eb2, wsd_t, bsd, wsn_t,
        wsu_t, bsu, whd_t[0].copy(), whs_t[0].copy())
    deg2 = _sc_deg(dst3d)

    for l in range(NL):
        pre = _sc_gather(a, b, dst3d, src3d)
        m = _edge_mlp(pre, w2_t[l], mb2[l])
        partials = _sc_scatter(m, dst3d)
        hn, gsum, gsq = _update(h, partials, deg2, su_all, l, uh_t[l],
                                ua_t[l], u2_t[l], ub2[l], batch2d)
        if l < NL - 1:
            h, a, b = _norm_proj(hn, gsum, gsq, cb, batch2d, whd_t[l + 1],
                                 whs_t[l + 1], sd_all, ssn_all, l + 1)
        else:
            h = _norm_last(hn, gsum, gsq, cb, batch2d)

    return _decoder(h, x, dt2d, w1d, b1d, w2d, b2s)


# speedup vs baseline: 7.9160x; 1.0235x over previous
import functools

import numpy as np
import jax
import jax.numpy as jnp
from jax import lax
from jax.experimental import pallas as pl
from jax.experimental.pallas import tpu as pltpu
from jax.experimental.pallas import tpu_sc as plsc

N = 10000
E = 320000
ED = 128
NL = 6
NG = 16
TW = 25
T_MAX = 4.0
EPS = 1e-5

NC = 2
NS = 16
NW = NC * NS
EPW = E // NW
R = 80
J = EPW // R
NCHUNK = E // R
NPAD = 10240
RPT = NPAD // NS

BN = 1000
GN = N // BN
BE = 2000
GE = E // BE

_SC_MESH = plsc.VectorSubcoreMesh(core_axis_name="c", subcore_axis_name="s")


def _silu(v):
    return v * jax.nn.sigmoid(v)


def _add3_rows(out_v, a_v, b_v):
    def row(r, carry):
        for jj in range(ED // 16):
            sl = pl.ds(jj * 16, 16)
            out_v[r, sl] = a_v[r, sl] + b_v[r, sl]
        return carry

    lax.fori_loop(0, R, row, 0)


def _sc_gather_common(jc, a_hbm, b_hbm, dst_hbm, src_hbm, out_hbm,
                      idxd_v, idxs_v, a0, b0, a1, b1, o0, o1,
                      sa0, sb0, sa1, sb1, sw0, sw1):
    c = lax.axis_index("c")
    s = lax.axis_index("s")
    wid = s * NC + c
    pltpu.sync_copy(dst_hbm.at[wid], idxd_v)
    pltpu.sync_copy(src_hbm.at[wid], idxs_v)
    base = wid * jc

    def fire(ck_local, ra, rb, sa, sb):
        pltpu.async_copy(a_hbm.at[idxd_v.at[ck_local]], ra, sa)
        pltpu.async_copy(b_hbm.at[idxs_v.at[ck_local]], rb, sb)

    def drain(ck_local, first, ra, rb, ro, sa, sb, sw):
        pltpu.make_async_copy(a_hbm.at[idxd_v.at[ck_local]], ra, sa).wait()
        pltpu.make_async_copy(b_hbm.at[idxs_v.at[ck_local]], rb, sb).wait()

        @pl.when(jnp.logical_not(first))
        def _():
            pltpu.make_async_copy(ro, out_hbm.at[base + ck_local],
                                  sw).wait()

        _add3_rows(ro, ra, rb)
        pltpu.async_copy(ro, out_hbm.at[base + ck_local], sw)

    fire(0, a0, b0, sa0, sb0)

    def step(i, carry):
        fire(2 * i + 1, a1, b1, sa1, sb1)
        drain(2 * i, i == 0, a0, b0, o0, sa0, sb0, sw0)
        fire(2 * i + 2, a0, b0, sa0, sb0)
        drain(2 * i + 1, i == 0, a1, b1, o1, sa1, sb1, sw1)
        return carry

    lax.fori_loop(0, (jc - 1) // 2, step, 0)
    drain(jc - 1, jc == 1, a0, b0, o0, sa0, sb0, sw0)
    pltpu.make_async_copy(o0, out_hbm.at[base + jc - 1], sw0).wait()
    if jc > 1:
        pltpu.make_async_copy(o1, out_hbm.at[base + jc - 2], sw1).wait()


def _gather_scratch(jc):
    return [
        pltpu.VMEM((jc, R), jnp.int32),
        pltpu.VMEM((jc, R), jnp.int32),
        pltpu.VMEM((R, ED), jnp.float32),
        pltpu.VMEM((R, ED), jnp.float32),
        pltpu.VMEM((R, ED), jnp.float32),
        pltpu.VMEM((R, ED), jnp.float32),
        pltpu.VMEM((R, ED), jnp.float32),
        pltpu.VMEM((R, ED), jnp.float32),
        pltpu.SemaphoreType.DMA,
        pltpu.SemaphoreType.DMA,
        pltpu.SemaphoreType.DMA,
        pltpu.SemaphoreType.DMA,
        pltpu.SemaphoreType.DMA,
        pltpu.SemaphoreType.DMA,
    ]


def _sc_gather(a, b, dst3d, src3d):
    jc = dst3d.shape[1]
    nch = NW * jc

    def body(a_hbm, b_hbm, dst_hbm, src_hbm, out_hbm,
             idxd_v, idxs_v, a0, b0, a1, b1, o0, o1,
             sa0, sb0, sa1, sb1, sw0, sw1):
        _sc_gather_common(jc, a_hbm, b_hbm, dst_hbm, src_hbm, out_hbm,
                          idxd_v, idxs_v, a0, b0, a1, b1, o0, o1,
                          sa0, sb0, sa1, sb1, sw0, sw1)

    f = pl.kernel(
        body,
        out_type=jax.ShapeDtypeStruct((nch, R, ED), jnp.float32),
        mesh=_SC_MESH,
        scratch_types=_gather_scratch(jc),
    )
    return f(a, b, dst3d, src3d)


def _scat_subloop(jc, wid, m_hbm, idx_v, m0_v, m1_v, acc_sh, s0, s1,
                  ss0, ss1):
    base = wid * jc
    pltpu.async_copy(m_hbm.at[base], m0_v, s0)

    def step(i, carry):
        @pl.when(i > 0)
        def _():
            pltpu.make_async_copy(m1_v, acc_sh.at[idx_v.at[2 * i - 1]],
                                  ss1).wait()

        pltpu.async_copy(m_hbm.at[base + 2 * i + 1], m1_v, s1)
        pltpu.make_async_copy(m_hbm.at[base + 2 * i], m0_v, s0).wait()
        pltpu.async_copy(m0_v, acc_sh.at[idx_v.at[2 * i]], ss0, add=True)
        pltpu.make_async_copy(m_hbm.at[base + 2 * i + 1], m1_v, s1).wait()
        pltpu.make_async_copy(m0_v, acc_sh.at[idx_v.at[2 * i]], ss0).wait()
        pltpu.async_copy(m_hbm.at[base + 2 * i + 2], m0_v, s0)
        pltpu.async_copy(m1_v, acc_sh.at[idx_v.at[2 * i + 1]], ss1,
                         add=True)
        return carry

    lax.fori_loop(0, (jc - 1) // 2, step, 0)
    pltpu.make_async_copy(m_hbm.at[base + jc - 1], m0_v, s0).wait()
    pltpu.async_copy(m0_v, acc_sh.at[idx_v.at[jc - 1]], ss0, add=True)
    pltpu.make_async_copy(m0_v, acc_sh.at[idx_v.at[jc - 1]], ss0).wait()
    pltpu.make_async_copy(m1_v, acc_sh.at[idx_v.at[jc - 2]], ss1).wait()


def _sc_scatter3(m1, m2, m3, d1, d2, d3):
    jcs = (d1.shape[1], d2.shape[1], d3.shape[1])

    def body(m1_hbm, m2_hbm, m3_hbm, d1_hbm, d2_hbm, d3_hbm, out_hbm,
             i1_v, i2_v, i3_v, m0_v, m1_v, acc_sh, s0, s1, ss0, ss1):
        c = lax.axis_index("c")
        s = lax.axis_index("s")
        wid = s * NC + c

        def zrow(r, carry):
            for jj in range(ED // 16):
                m0_v[r, pl.ds(jj * 16, 16)] = jnp.zeros((16,), jnp.float32)
            return carry

        lax.fori_loop(0, R, zrow, 0)
        for q in range(RPT // R):
            pltpu.sync_copy(m0_v, acc_sh.at[pl.ds(s * RPT + q * R, R)])
        pltpu.sync_copy(d1_hbm.at[wid], i1_v)
        pltpu.sync_copy(d2_hbm.at[wid], i2_v)
        pltpu.sync_copy(d3_hbm.at[wid], i3_v)
        plsc.subcore_barrier()
        for mh, iv, jc in ((m1_hbm, i1_v, jcs[0]), (m2_hbm, i2_v, jcs[1]),
                           (m3_hbm, i3_v, jcs[2])):
            _scat_subloop(jc, wid, mh, iv, m0_v, m1_v, acc_sh, s0, s1,
                          ss0, ss1)
        plsc.subcore_barrier()
        pltpu.sync_copy(acc_sh.at[pl.ds(s * RPT, RPT)],
                        out_hbm.at[c, pl.ds(s * RPT, RPT)])

    f = pl.kernel(
        body,
        out_type=jax.ShapeDtypeStruct((NC, NPAD, ED), jnp.float32),
        mesh=_SC_MESH,
        scratch_types=[
            pltpu.VMEM((jcs[0], R), jnp.int32),
            pltpu.VMEM((jcs[1], R), jnp.int32),
            pltpu.VMEM((jcs[2], R), jnp.int32),
            pltpu.VMEM((R, ED), jnp.float32),
            pltpu.VMEM((R, ED), jnp.float32),
            pltpu.VMEM_SHARED((NPAD, ED), jnp.float32),
            pltpu.SemaphoreType.DMA,
            pltpu.SemaphoreType.DMA,
            pltpu.SemaphoreType.DMA,
            pltpu.SemaphoreType.DMA,
        ],
    )
    return f(m1, m2, m3, d1, d2, d3)


def _sc_deg_body(dst_hbm, out_hbm, idx_v, ones_v, z_v, acc_sh):
    c = lax.axis_index("c")
    s = lax.axis_index("s")
    wid = s * NC + c

    def frow(r, carry):
        for jj in range(ED // 16):
            ones_v[r, pl.ds(jj * 16, 16)] = jnp.ones((16,), jnp.float32)
            z_v[r, pl.ds(jj * 16, 16)] = jnp.zeros((16,), jnp.float32)
        return carry

    lax.fori_loop(0, R, frow, 0)
    for q in range(RPT // R):
        pltpu.sync_copy(z_v, acc_sh.at[pl.ds(s * RPT + q * R, R)])
    pltpu.sync_copy(dst_hbm.at[wid], idx_v)
    plsc.subcore_barrier()

    def step(j, carry):
        pltpu.sync_copy(ones_v, acc_sh.at[idx_v.at[j]], add=True)
        return carry

    lax.fori_loop(0, J, step, 0)
    plsc.subcore_barrier()
    pltpu.sync_copy(acc_sh.at[pl.ds(s * RPT, RPT)],
                    out_hbm.at[c, pl.ds(s * RPT, RPT)])


def _sc_deg(dst3d):
    f = pl.kernel(
        _sc_deg_body,
        out_type=jax.ShapeDtypeStruct((NC, NPAD, ED), jnp.float32),
        mesh=_SC_MESH,
        scratch_types=[
            pltpu.VMEM((J, R), jnp.int32),
            pltpu.VMEM((R, ED), jnp.float32),
            pltpu.VMEM((R, ED), jnp.float32),
            pltpu.VMEM_SHARED((NPAD, ED), jnp.float32),
        ],
    )
    return f(dst3d)


def _posmax_body(pos_ref, out_ref):
    out_ref[0, 0] = jnp.max(pos_ref[...])


def _posmax(pos_pad):
    return pl.pallas_call(
        _posmax_body,
        out_shape=jax.ShapeDtypeStruct((1, 1), jnp.float32),
        in_specs=[pl.BlockSpec(pos_pad.shape, lambda: (0, 0))],
        out_specs=pl.BlockSpec((1, 1), lambda: (0, 0),
                               memory_space=pltpu.SMEM),
    )(pos_pad)


def _enc_body(posmax_ref, inp_ref, batch_ref, ew1t_ref, eb1_ref, ew2t_ref,
              eb2_ref, wsd_ref, bsd_ref, wsn_ref, wsu_ref, bsu_ref,
              whd0t_ref, whs0t_ref,
              h0_ref, a0_ref, b0_ref, sd_ref, ssn_ref, su_ref, cb_ref):
    col = lax.broadcasted_iota(jnp.int32, (1, 30), 1)
    inv_pm = 1.0 / posmax_ref[0, 0]
    scale = jnp.where(col == 25, inv_pm,
                      jnp.where(col == 26, 1.0 / T_MAX, 1.0))
    xb = inp_ref[...] * scale
    h1 = _silu(jnp.dot(xb, ew1t_ref[...],
                       preferred_element_type=jnp.float32) + eb1_ref[...])
    h0 = _silu(jnp.dot(h1, ew2t_ref[...],
                       preferred_element_type=jnp.float32) + eb2_ref[...])
    sd = jnp.dot(xb, wsd_ref[...],
                 preferred_element_type=jnp.float32) + bsd_ref[...]
    ssn = jnp.dot(xb, wsn_ref[...], preferred_element_type=jnp.float32)
    su = jnp.dot(xb, wsu_ref[...],
                 preferred_element_type=jnp.float32) + bsu_ref[...]
    h0_ref[...] = h0
    a0_ref[...] = jnp.dot(h0, whd0t_ref[...],
                          preferred_element_type=jnp.float32) + sd[:, :ED]
    b0_ref[...] = jnp.dot(h0, whs0t_ref[...],
                          preferred_element_type=jnp.float32) + ssn[:, :ED]
    sd_ref[...] = sd
    ssn_ref[...] = ssn
    su_ref[...] = su
    mask = (batch_ref[...] ==
            lax.broadcasted_iota(jnp.int32, (1, NG), 1)).astype(jnp.float32)
    part = jnp.sum(mask, axis=0, keepdims=True)

    @pl.when(pl.program_id(0) == 0)
    def _():
        cb_ref[...] = jnp.zeros_like(cb_ref)

    cb_ref[...] += part


def _encoder(posmax, inp30, batch2d, ew1t, eb1, ew2t, eb2, wsd, bsd, wsn,
             wsu, bsu, whd0t, whs0t):
    SD = NL * ED
    full = lambda shp: pl.BlockSpec(shp, lambda i: (0, 0))
    return pl.pallas_call(
        _enc_body,
        grid=(GN,),
        out_shape=(
            jax.ShapeDtypeStruct((N, ED), jnp.float32),
            jax.ShapeDtypeStruct((N, ED), jnp.float32),
            jax.ShapeDtypeStruct((N, ED), jnp.float32),
            jax.ShapeDtypeStruct((N, SD), jnp.float32),
            jax.ShapeDtypeStruct((N, SD), jnp.float32),
            jax.ShapeDtypeStruct((N, SD), jnp.float32),
            jax.ShapeDtypeStruct((1, NG), jnp.float32),
        ),
        in_specs=[
            pl.BlockSpec(memory_space=pltpu.SMEM),
            pl.BlockSpec((BN, 30), lambda i: (i, 0)),
            pl.BlockSpec((BN, 1), lambda i: (i, 0)),
            full((30, ED)), full((1, ED)), full((ED, ED)), full((1, ED)),
            full((30, SD)), full((1, SD)), full((30, SD)), full((30, SD)),
            full((1, SD)), full((ED, ED)), full((ED, ED)),
        ],
        out_specs=(
            pl.BlockSpec((BN, ED), lambda i: (i, 0)),
            pl.BlockSpec((BN, ED), lambda i: (i, 0)),
            pl.BlockSpec((BN, ED), lambda i: (i, 0)),
            pl.BlockSpec((BN, SD), lambda i: (i, 0)),
            pl.BlockSpec((BN, SD), lambda i: (i, 0)),
            pl.BlockSpec((BN, SD), lambda i: (i, 0)),
            pl.BlockSpec((1, NG), lambda i: (0, 0)),
        ),
    )(posmax, inp30, batch2d, ew1t, eb1, ew2t, eb2, wsd, bsd, wsn, wsu,
      bsu, whd0t, whs0t)


def _edge_mlp(pre, w2t, b2):
    nch = pre.shape[0]
    bc = 16
    be = bc * R

    def body(pre_ref, w2t_ref, b2_ref, m_ref):
        sv = _silu(pre_ref[...].reshape(be, ED))
        mv = _silu(jnp.dot(sv.astype(jnp.bfloat16),
                           w2t_ref[...].astype(jnp.bfloat16),
                           preferred_element_type=jnp.float32)
                   + b2_ref[...])
        m_ref[...] = mv.reshape(bc, R, ED)

    return pl.pallas_call(
        body,
        grid=(nch // bc,),
        out_shape=jax.ShapeDtypeStruct((nch, R, ED), jnp.float32),
        in_specs=[
            pl.BlockSpec((bc, R, ED), lambda i: (i, 0, 0)),
            pl.BlockSpec((ED, ED), lambda i: (0, 0)),
            pl.BlockSpec((1, ED), lambda i: (0, 0)),
        ],
        out_specs=pl.BlockSpec((bc, R, ED), lambda i: (i, 0, 0)),
    )(pre, w2t, b2)


def _upd_body(h_ref, p_ref, deg_ref, su_ref, uht_ref, uat_ref, u2t_ref,
              ub2_ref, batch_ref, hn_ref, gsum_ref, gsq_ref):
    psum = p_ref[0] + p_ref[1]
    degv = jnp.maximum(deg_ref[0, :, 0:1] + deg_ref[1, :, 0:1], 1.0)
    agg = psum * (1.0 / degv)
    bf = jnp.bfloat16
    u1 = _silu(jnp.dot(h_ref[...].astype(bf), uht_ref[...].astype(bf),
                       preferred_element_type=jnp.float32)
               + jnp.dot(agg.astype(bf), uat_ref[...].astype(bf),
                         preferred_element_type=jnp.float32)
               + su_ref[...])
    up = _silu(jnp.dot(u1.astype(bf), u2t_ref[...].astype(bf),
                       preferred_element_type=jnp.float32) + ub2_ref[...])
    hn = h_ref[...] + up
    hn_ref[...] = hn
    mask = (batch_ref[...] ==
            lax.broadcasted_iota(jnp.int32, (1, NG), 1)).astype(jnp.float32)
    gs = lax.dot_general(mask, hn, (((0,), (0,)), ((), ())),
                         preferred_element_type=jnp.float32)
    gq = lax.dot_general(mask, hn * hn, (((0,), (0,)), ((), ())),
                         preferred_element_type=jnp.float32)

    @pl.when(pl.program_id(0) == 0)
    def _():
        gsum_ref[...] = jnp.zeros_like(gsum_ref)
        gsq_ref[...] = jnp.zeros_like(gsq_ref)

    gsum_ref[...] += gs
    gsq_ref[...] += gq


def _update(h, partials, deg2, su_all, layer, uht, uat, u2t, ub2, batch2d):
    return pl.pallas_call(
        _upd_body,
        grid=(GN,),
        out_shape=(
            jax.ShapeDtypeStruct((N, ED), jnp.float32),
            jax.ShapeDtypeStruct((NG, ED), jnp.float32),
            jax.ShapeDtypeStruct((NG, ED), jnp.float32),
        ),
        in_specs=[
            pl.BlockSpec((BN, ED), lambda i: (i, 0)),
            pl.BlockSpec((NC, BN, ED), lambda i: (0, i, 0)),
            pl.BlockSpec((NC, BN, ED), lambda i: (0, i, 0)),
            pl.BlockSpec((BN, ED), lambda i, L=layer: (i, L)),
            pl.BlockSpec((ED, ED), lambda i: (0, 0)),
            pl.BlockSpec((ED, ED), lambda i: (0, 0)),
            pl.BlockSpec((ED, ED), lambda i: (0, 0)),
            pl.BlockSpec((1, ED), lambda i: (0, 0)),
            pl.BlockSpec((BN, 1), lambda i: (i, 0)),
        ],
        out_specs=(
            pl.BlockSpec((BN, ED), lambda i: (i, 0)),
            pl.BlockSpec((NG, ED), lambda i: (0, 0)),
            pl.BlockSpec((NG, ED), lambda i: (0, 0)),
        ),
    )(h, partials, deg2, su_all, uht, uat, u2t, ub2, batch2d)


def _norm_body_proj(hn_ref, gsum_ref, gsq_ref, cb_ref, batch_ref, whdt_ref,
                    whst_ref, sd_ref, ssn_ref, h_ref, a_ref, b_ref):
    hb = _norm_common(hn_ref, gsum_ref, gsq_ref, cb_ref, batch_ref)
    h_ref[...] = hb
    a_ref[...] = jnp.dot(hb, whdt_ref[...],
                         preferred_element_type=jnp.float32) + sd_ref[...]
    b_ref[...] = jnp.dot(hb, whst_ref[...],
                         preferred_element_type=jnp.float32) + ssn_ref[...]


def _norm_body_last(hn_ref, gsum_ref, gsq_ref, cb_ref, batch_ref, h_ref):
    h_ref[...] = _norm_common(hn_ref, gsum_ref, gsq_ref, cb_ref, batch_ref)


def _norm_common(hn_ref, gsum_ref, gsq_ref, cb_ref, batch_ref):
    cbm = jnp.maximum(cb_ref[...], 1.0)
    mask = (batch_ref[...] ==
            lax.broadcasted_iota(jnp.int32, (1, NG), 1)).astype(jnp.float32)
    maskc = mask * (1.0 / cbm)
    meanr = jnp.dot(maskc, gsum_ref[...], preferred_element_type=jnp.float32)
    eh2r = jnp.dot(maskc, gsq_ref[...], preferred_element_type=jnp.float32)
    varr = jnp.maximum(eh2r - meanr * meanr, 0.0)
    return (hn_ref[...] - meanr) * lax.rsqrt(varr + EPS)


def _norm_proj(hn, gsum, gsq, cb, batch2d, whdt, whst, sd_all, ssn_all,
               layer_next):
    return pl.pallas_call(
        _norm_body_proj,
        grid=(GN,),
        out_shape=(
            jax.ShapeDtypeStruct((N, ED), jnp.float32),
            jax.ShapeDtypeStruct((N, ED), jnp.float32),
            jax.ShapeDtypeStruct((N, ED), jnp.float32),
        ),
        in_specs=[
            pl.BlockSpec((BN, ED), lambda i: (i, 0)),
            pl.BlockSpec((NG, ED), lambda i: (0, 0)),
            pl.BlockSpec((NG, ED), lambda i: (0, 0)),
            pl.BlockSpec((1, NG), lambda i: (0, 0)),
            pl.BlockSpec((BN, 1), lambda i: (i, 0)),
            pl.BlockSpec((ED, ED), lambda i: (0, 0)),
            pl.BlockSpec((ED, ED), lambda i: (0, 0)),
            pl.BlockSpec((BN, ED), lambda i, L=layer_next: (i, L)),
            pl.BlockSpec((BN, ED), lambda i, L=layer_next: (i, L)),
        ],
        out_specs=(
            pl.BlockSpec((BN, ED), lambda i: (i, 0)),
            pl.BlockSpec((BN, ED), lambda i: (i, 0)),
            pl.BlockSpec((BN, ED), lambda i: (i, 0)),
        ),
    )(hn, gsum, gsq, cb, batch2d, whdt, whst, sd_all, ssn_all)


def _norm_last(hn, gsum, gsq, cb, batch2d):
    return pl.pallas_call(
        _norm_body_last,
        grid=(GN,),
        out_shape=jax.ShapeDtypeStruct((N, ED), jnp.float32),
        in_specs=[
            pl.BlockSpec((BN, ED), lambda i: (i, 0)),
            pl.BlockSpec((NG, ED), lambda i: (0, 0)),
            pl.BlockSpec((NG, ED), lambda i: (0, 0)),
            pl.BlockSpec((1, NG), lambda i: (0, 0)),
            pl.BlockSpec((BN, 1), lambda i: (i, 0)),
        ],
        out_specs=pl.BlockSpec((BN, ED), lambda i: (i, 0)),
    )(hn, gsum, gsq, cb, batch2d)


def _dec_body(h_ref, x_ref, dt_ref, w1d_ref, b1d_ref, w2d_ref, b2s_ref,
              out_ref):
    z1 = _silu(jnp.dot(h_ref[...], w1d_ref[...],
                       preferred_element_type=jnp.float32) + b1d_ref[...])
    z2 = jnp.dot(z1, w2d_ref[...],
                 preferred_element_type=jnp.float32) + b2s_ref[0, 0]
    steps = (lax.broadcasted_iota(jnp.int32, (1, TW), 1) + 1
             ).astype(jnp.float32)
    dtv = dt_ref[0, 0] * steps
    out_ref[...] = x_ref[:, TW - 1:TW] + dtv * z2


def _decoder(h, x, dt2d, w1d, b1d, w2d, b2s):
    return pl.pallas_call(
        _dec_body,
        grid=(GN,),
        out_shape=jax.ShapeDtypeStruct((N, TW), jnp.float32),
        in_specs=[
            pl.BlockSpec((BN, ED), lambda i: (i, 0)),
            pl.BlockSpec((BN, TW), lambda i: (i, 0)),
            pl.BlockSpec(memory_space=pltpu.SMEM),
            pl.BlockSpec((ED, 8 * 38), lambda i: (0, 0)),
            pl.BlockSpec((1, 8 * 38), lambda i: (0, 0)),
            pl.BlockSpec((8 * 38, TW), lambda i: (0, 0)),
            pl.BlockSpec(memory_space=pltpu.SMEM),
        ],
        out_specs=pl.BlockSpec((BN, TW), lambda i: (i, 0)),
    )(h, x, dt2d, w1d, b1d, w2d, b2s)


def _densify_convs(conv1_w, conv1_b, conv2_w, conv2_b):
    o_i, p_i, k_i = np.meshgrid(np.arange(8), np.arange(38), np.arange(16),
                                indexing="ij")
    rows1 = (3 * p_i + k_i).reshape(-1)
    cols1 = (o_i * 38 + p_i).reshape(-1)
    w1d = jnp.zeros((ED, 8 * 38), jnp.float32).at[rows1, cols1].set(
        conv1_w[o_i.reshape(-1), 0, k_i.reshape(-1)])
    b1d = jnp.repeat(conv1_b, 38).reshape(1, 8 * 38)
    o_i, q_i, d_i = np.meshgrid(np.arange(8), np.arange(TW), np.arange(14),
                                indexing="ij")
    rows2 = (o_i * 38 + q_i + d_i).reshape(-1)
    cols2 = q_i.reshape(-1)
    w2d = jnp.zeros((8 * 38, TW), jnp.float32).at[rows2, cols2].set(
        conv2_w[0, o_i.reshape(-1), d_i.reshape(-1)])
    b2s = conv2_b.reshape(1, 1)
    return w1d, b1d, w2d, b2s


def kernel(x, pos, t, vars_abc, dt, enc_w1, enc_b1, enc_w2, enc_b2,
           msg1_w, msg1_b, msg2_w, msg2_b, upd1_w, upd1_b, upd2_w, upd2_b,
           conv1_w, conv1_b, conv2_w, conv2_b, edge_index, batch):
    f32 = jnp.float32
    SD = NL * ED

    inp30 = jnp.concatenate([x, pos, t, vars_abc], axis=-1)
    batch2d = batch.reshape(N, 1).astype(jnp.int32)
    srcv = edge_index[0].astype(jnp.int32)
    dstv = edge_index[1].astype(jnp.int32)
    dst3d = dstv.reshape(NW, J, R)
    jsl = (41, 41, 43)
    esl = [NW * jc * R for jc in jsl]
    o1, o2 = esl[0], esl[0] + esl[1]
    srcS = (srcv[:o1].reshape(NW, jsl[0], R),
            srcv[o1:o2].reshape(NW, jsl[1], R),
            srcv[o2:].reshape(NW, jsl[2], R))
    dstS = (dstv[:o1].reshape(NW, jsl[0], R),
            dstv[o1:o2].reshape(NW, jsl[1], R),
            dstv[o2:].reshape(NW, jsl[2], R))
    pos_pad = jnp.concatenate(
        [pos.reshape(-1), jnp.zeros((240,), f32)]).reshape(80, 128)

    whd_t = jnp.transpose(msg1_w[:, :, :ED], (0, 2, 1))
    whs_t = jnp.transpose(msg1_w[:, :, ED:2 * ED], (0, 2, 1))
    wx = msg1_w[:, :, 2 * ED:2 * ED + TW]
    wp = msg1_w[:, :, 2 * ED + TW:2 * ED + TW + 1]
    wv = msg1_w[:, :, 2 * ED + TW + 1:]
    wsd = jnp.concatenate([wx, wp, wv], axis=2)
    wsd_t = wsd.reshape(SD, 30).T
    bsd = msg1_b.reshape(1, SD)
    wsn = -jnp.concatenate([wx, wp, jnp.zeros((NL, ED, 4), f32)], axis=2)
    wsn_t = wsn.reshape(SD, 30).T
    uv = upd1_w[:, :, 2 * ED:]
    wsu = jnp.concatenate([jnp.zeros((NL, ED, 26), f32), uv], axis=2)
    wsu_t = wsu.reshape(SD, 30).T
    bsu = upd1_b.reshape(1, SD)
    uh_t = jnp.transpose(upd1_w[:, :, :ED], (0, 2, 1))
    ua_t = jnp.transpose(upd1_w[:, :, ED:2 * ED], (0, 2, 1))
    w2_t = jnp.transpose(msg2_w, (0, 2, 1))
    u2_t = jnp.transpose(upd2_w, (0, 2, 1))
    ew1t = enc_w1.T
    ew2t = enc_w2.T
    eb1 = enc_b1.reshape(1, ED)
    eb2 = enc_b2.reshape(1, ED)
    mb2 = msg2_b.reshape(NL, 1, ED)
    ub2 = upd2_b.reshape(NL, 1, ED)
    w1d, b1d, w2d, b2s = _densify_convs(conv1_w, conv1_b, conv2_w, conv2_b)
    dt2d = dt.reshape(1, 1)

    posmax = _posmax(pos_pad)
    h, a, b, sd_all, ssn_all, su_all, cb = _encoder(
        posmax, inp30, batch2d, ew1t, eb1, ew2t, eb2, wsd_t, bsd, wsn_t,
        wsu_t, bsu, whd_t[0].copy(), whs_t[0].copy())
    deg2 = _sc_deg(dst3d)

    for l in range(NL):
        pre1 = _sc_gather(a, b, dstS[0], srcS[0])
        pre2 = _sc_gather(a, b, dstS[1], srcS[1])
        m1 = _edge_mlp(pre1, w2_t[l], mb2[l])
        pre3 = _sc_gather(a, b, dstS[2], srcS[2])
        m2 = _edge_mlp(pre2, w2_t[l], mb2[l])
        m3 = _edge_mlp(pre3, w2_t[l], mb2[l])
        partials = _sc_scatter3(m1, m2, m3, dstS[0], dstS[1], dstS[2])
        hn, gsum, gsq = _update(h, partials, deg2, su_all, l, uh_t[l],
                                ua_t[l], u2_t[l], ub2[l], batch2d)
        if l < NL - 1:
            h, a, b = _norm_proj(hn, gsum, gsq, cb, batch2d, whd_t[l + 1],
                                 whs_t[l + 1], sd_all, ssn_all, l + 1)
        else:
            h = _norm_last(hn, gsum, gsq, cb, batch2d)

    return _decoder(h, x, dt2d, w1d, b1d, w2d, b2s)


# speedup vs baseline: 8.5804x; 1.0839x over previous
import functools

import numpy as np
import jax
import jax.numpy as jnp
from jax import lax
from jax.experimental import pallas as pl
from jax.experimental.pallas import tpu as pltpu
from jax.experimental.pallas import tpu_sc as plsc

N = 10000
E = 320000
ED = 128
NL = 6
NG = 16
TW = 25
T_MAX = 4.0
EPS = 1e-5

NC = 2
NS = 16
NW = NC * NS
EPW = E // NW
R = 80
J = EPW // R
NCHUNK = E // R
NPAD = 10240
RPT = NPAD // NS

BN = 1000
GN = N // BN
BE = 2000
GE = E // BE

_SC_MESH = plsc.VectorSubcoreMesh(core_axis_name="c", subcore_axis_name="s")


def _silu(v):
    return v * jax.nn.sigmoid(v)


def _add3_rows(out_v, a_v, b_v):
    def row(r, carry):
        for jj in range(ED // 16):
            sl = pl.ds(jj * 16, 16)
            out_v[r, sl] = a_v[r, sl] + b_v[r, sl]
        return carry

    lax.fori_loop(0, R, row, 0)


def _sc_gather_common(jc, a_hbm, b_hbm, dst_hbm, src_hbm, out_hbm,
                      idxd_v, idxs_v, a0, b0, a1, b1, o0, o1,
                      sa0, sb0, sa1, sb1, sw0, sw1):
    c = lax.axis_index("c")
    s = lax.axis_index("s")
    wid = s * NC + c
    pltpu.sync_copy(dst_hbm.at[wid], idxd_v)
    pltpu.sync_copy(src_hbm.at[wid], idxs_v)
    base = wid * jc

    def fire(ck_local, ra, rb, sa, sb):
        pltpu.async_copy(a_hbm.at[idxd_v.at[ck_local]], ra, sa)
        pltpu.async_copy(b_hbm.at[idxs_v.at[ck_local]], rb, sb)

    def drain(ck_local, first, ra, rb, ro, sa, sb, sw):
        pltpu.make_async_copy(a_hbm.at[idxd_v.at[ck_local]], ra, sa).wait()
        pltpu.make_async_copy(b_hbm.at[idxs_v.at[ck_local]], rb, sb).wait()

        @pl.when(jnp.logical_not(first))
        def _():
            pltpu.make_async_copy(ro, out_hbm.at[base + ck_local],
                                  sw).wait()

        _add3_rows(ro, ra, rb)
        pltpu.async_copy(ro, out_hbm.at[base + ck_local], sw)

    fire(0, a0, b0, sa0, sb0)

    def step(i, carry):
        fire(2 * i + 1, a1, b1, sa1, sb1)
        drain(2 * i, i == 0, a0, b0, o0, sa0, sb0, sw0)
        fire(2 * i + 2, a0, b0, sa0, sb0)
        drain(2 * i + 1, i == 0, a1, b1, o1, sa1, sb1, sw1)
        return carry

    lax.fori_loop(0, (jc - 1) // 2, step, 0)
    drain(jc - 1, jc == 1, a0, b0, o0, sa0, sb0, sw0)
    pltpu.make_async_copy(o0, out_hbm.at[base + jc - 1], sw0).wait()
    if jc > 1:
        pltpu.make_async_copy(o1, out_hbm.at[base + jc - 2], sw1).wait()


def _gather_scratch(jc):
    return [
        pltpu.VMEM((jc, R), jnp.int32),
        pltpu.VMEM((jc, R), jnp.int32),
        pltpu.VMEM((R, ED), jnp.float32),
        pltpu.VMEM((R, ED), jnp.float32),
        pltpu.VMEM((R, ED), jnp.float32),
        pltpu.VMEM((R, ED), jnp.float32),
        pltpu.VMEM((R, ED), jnp.float32),
        pltpu.VMEM((R, ED), jnp.float32),
        pltpu.SemaphoreType.DMA,
        pltpu.SemaphoreType.DMA,
        pltpu.SemaphoreType.DMA,
        pltpu.SemaphoreType.DMA,
        pltpu.SemaphoreType.DMA,
        pltpu.SemaphoreType.DMA,
    ]


def _sc_gather(a, b, dst3d, src3d):
    jc = dst3d.shape[1]
    nch = NW * jc

    def body(a_hbm, b_hbm, dst_hbm, src_hbm, out_hbm,
             idxd_v, idxs_v, a0, b0, a1, b1, o0, o1,
             sa0, sb0, sa1, sb1, sw0, sw1):
        _sc_gather_common(jc, a_hbm, b_hbm, dst_hbm, src_hbm, out_hbm,
                          idxd_v, idxs_v, a0, b0, a1, b1, o0, o1,
                          sa0, sb0, sa1, sb1, sw0, sw1)

    f = pl.kernel(
        body,
        out_type=jax.ShapeDtypeStruct((nch, R, ED), jnp.float32),
        mesh=_SC_MESH,
        scratch_types=_gather_scratch(jc),
    )
    return f(a, b, dst3d, src3d)


def _scat_subloop(jc, wid, m_hbm, idx_v, m0_v, m1_v, acc_sh, s0, s1,
                  ss0, ss1):
    base = wid * jc
    pltpu.async_copy(m_hbm.at[base], m0_v, s0)

    def step(i, carry):
        @pl.when(i > 0)
        def _():
            pltpu.make_async_copy(m1_v, acc_sh.at[idx_v.at[2 * i - 1]],
                                  ss1).wait()

        pltpu.async_copy(m_hbm.at[base + 2 * i + 1], m1_v, s1)
        pltpu.make_async_copy(m_hbm.at[base + 2 * i], m0_v, s0).wait()
        pltpu.async_copy(m0_v, acc_sh.at[idx_v.at[2 * i]], ss0, add=True)
        pltpu.make_async_copy(m_hbm.at[base + 2 * i + 1], m1_v, s1).wait()
        pltpu.make_async_copy(m0_v, acc_sh.at[idx_v.at[2 * i]], ss0).wait()
        pltpu.async_copy(m_hbm.at[base + 2 * i + 2], m0_v, s0)
        pltpu.async_copy(m1_v, acc_sh.at[idx_v.at[2 * i + 1]], ss1,
                         add=True)
        return carry

    lax.fori_loop(0, (jc - 1) // 2, step, 0)
    pltpu.make_async_copy(m_hbm.at[base + jc - 1], m0_v, s0).wait()
    pltpu.async_copy(m0_v, acc_sh.at[idx_v.at[jc - 1]], ss0, add=True)
    pltpu.make_async_copy(m0_v, acc_sh.at[idx_v.at[jc - 1]], ss0).wait()
    pltpu.make_async_copy(m1_v, acc_sh.at[idx_v.at[jc - 2]], ss1).wait()


def _sc_scatter3(m1, m2, m3, d1, d2, d3):
    jcs = (d1.shape[1], d2.shape[1], d3.shape[1])

    def body(m1_hbm, m2_hbm, m3_hbm, d1_hbm, d2_hbm, d3_hbm, out_hbm,
             i1_v, i2_v, i3_v, m0_v, m1_v, acc_sh, s0, s1, ss0, ss1):
        c = lax.axis_index("c")
        s = lax.axis_index("s")
        wid = s * NC + c

        def zrow(r, carry):
            for jj in range(ED // 16):
                m0_v[r, pl.ds(jj * 16, 16)] = jnp.zeros((16,), jnp.float32)
            return carry

        lax.fori_loop(0, R, zrow, 0)
        for q in range(RPT // R):
            pltpu.sync_copy(m0_v, acc_sh.at[pl.ds(s * RPT + q * R, R)])
        pltpu.sync_copy(d1_hbm.at[wid], i1_v)
        pltpu.sync_copy(d2_hbm.at[wid], i2_v)
        pltpu.sync_copy(d3_hbm.at[wid], i3_v)
        plsc.subcore_barrier()
        for mh, iv, jc in ((m1_hbm, i1_v, jcs[0]), (m2_hbm, i2_v, jcs[1]),
                           (m3_hbm, i3_v, jcs[2])):
            _scat_subloop(jc, wid, mh, iv, m0_v, m1_v, acc_sh, s0, s1,
                          ss0, ss1)
        plsc.subcore_barrier()
        pltpu.sync_copy(acc_sh.at[pl.ds(s * RPT, RPT)],
                        out_hbm.at[c, pl.ds(s * RPT, RPT)])

    f = pl.kernel(
        body,
        out_type=jax.ShapeDtypeStruct((NC, NPAD, ED), jnp.float32),
        mesh=_SC_MESH,
        scratch_types=[
            pltpu.VMEM((jcs[0], R), jnp.int32),
            pltpu.VMEM((jcs[1], R), jnp.int32),
            pltpu.VMEM((jcs[2], R), jnp.int32),
            pltpu.VMEM((R, ED), jnp.float32),
            pltpu.VMEM((R, ED), jnp.float32),
            pltpu.VMEM_SHARED((NPAD, ED), jnp.float32),
            pltpu.SemaphoreType.DMA,
            pltpu.SemaphoreType.DMA,
            pltpu.SemaphoreType.DMA,
            pltpu.SemaphoreType.DMA,
        ],
    )
    return f(m1, m2, m3, d1, d2, d3)


def _sc_deg_body(dst_hbm, out_hbm, idx_v, ones_v, z_v, acc_sh):
    c = lax.axis_index("c")
    s = lax.axis_index("s")
    wid = s * NC + c

    def frow(r, carry):
        for jj in range(ED // 16):
            ones_v[r, pl.ds(jj * 16, 16)] = jnp.ones((16,), jnp.float32)
            z_v[r, pl.ds(jj * 16, 16)] = jnp.zeros((16,), jnp.float32)
        return carry

    lax.fori_loop(0, R, frow, 0)
    for q in range(RPT // R):
        pltpu.sync_copy(z_v, acc_sh.at[pl.ds(s * RPT + q * R, R)])
    pltpu.sync_copy(dst_hbm.at[wid], idx_v)
    plsc.subcore_barrier()

    def step(j, carry):
        pltpu.sync_copy(ones_v, acc_sh.at[idx_v.at[j]], add=True)
        return carry

    lax.fori_loop(0, J, step, 0)
    plsc.subcore_barrier()
    pltpu.sync_copy(acc_sh.at[pl.ds(s * RPT, RPT)],
                    out_hbm.at[c, pl.ds(s * RPT, RPT)])


def _sc_deg(dst3d):
    f = pl.kernel(
        _sc_deg_body,
        out_type=jax.ShapeDtypeStruct((NC, NPAD, ED), jnp.float32),
        mesh=_SC_MESH,
        scratch_types=[
            pltpu.VMEM((J, R), jnp.int32),
            pltpu.VMEM((R, ED), jnp.float32),
            pltpu.VMEM((R, ED), jnp.float32),
            pltpu.VMEM_SHARED((NPAD, ED), jnp.float32),
        ],
    )
    return f(dst3d)


def _posmax_body(pos_ref, out_ref):
    out_ref[0, 0] = jnp.max(pos_ref[...])


def _posmax(pos_pad):
    return pl.pallas_call(
        _posmax_body,
        out_shape=jax.ShapeDtypeStruct((1, 1), jnp.float32),
        in_specs=[pl.BlockSpec(pos_pad.shape, lambda: (0, 0))],
        out_specs=pl.BlockSpec((1, 1), lambda: (0, 0),
                               memory_space=pltpu.SMEM),
    )(pos_pad)


def _enc_body(posmax_ref, inp_ref, batch_ref, ew1t_ref, eb1_ref, ew2t_ref,
              eb2_ref, wsd_ref, bsd_ref, wsn_ref, wsu_ref, bsu_ref,
              whd0t_ref, whs0t_ref,
              h0_ref, a0_ref, b0_ref, sd_ref, ssn_ref, su_ref, cb_ref):
    col = lax.broadcasted_iota(jnp.int32, (1, 30), 1)
    inv_pm = 1.0 / posmax_ref[0, 0]
    scale = jnp.where(col == 25, inv_pm,
                      jnp.where(col == 26, 1.0 / T_MAX, 1.0))
    xb = inp_ref[...] * scale
    h1 = _silu(jnp.dot(xb, ew1t_ref[...],
                       preferred_element_type=jnp.float32) + eb1_ref[...])
    h0 = _silu(jnp.dot(h1, ew2t_ref[...],
                       preferred_element_type=jnp.float32) + eb2_ref[...])
    sd = jnp.dot(xb, wsd_ref[...],
                 preferred_element_type=jnp.float32) + bsd_ref[...]
    ssn = jnp.dot(xb, wsn_ref[...], preferred_element_type=jnp.float32)
    su = jnp.dot(xb, wsu_ref[...],
                 preferred_element_type=jnp.float32) + bsu_ref[...]
    h0_ref[...] = h0
    a0_ref[...] = jnp.dot(h0, whd0t_ref[...],
                          preferred_element_type=jnp.float32) + sd[:, :ED]
    b0_ref[...] = jnp.dot(h0, whs0t_ref[...],
                          preferred_element_type=jnp.float32) + ssn[:, :ED]
    sd_ref[...] = sd
    ssn_ref[...] = ssn
    su_ref[...] = su
    mask = (batch_ref[...] ==
            lax.broadcasted_iota(jnp.int32, (1, NG), 1)).astype(jnp.float32)
    part = jnp.sum(mask, axis=0, keepdims=True)

    @pl.when(pl.program_id(0) == 0)
    def _():
        cb_ref[...] = jnp.zeros_like(cb_ref)

    cb_ref[...] += part


def _encoder(posmax, inp30, batch2d, ew1t, eb1, ew2t, eb2, wsd, bsd, wsn,
             wsu, bsu, whd0t, whs0t):
    SD = NL * ED
    full = lambda shp: pl.BlockSpec(shp, lambda i: (0, 0))
    return pl.pallas_call(
        _enc_body,
        grid=(GN,),
        out_shape=(
            jax.ShapeDtypeStruct((N, ED), jnp.float32),
            jax.ShapeDtypeStruct((N, ED), jnp.float32),
            jax.ShapeDtypeStruct((N, ED), jnp.float32),
            jax.ShapeDtypeStruct((N, SD), jnp.float32),
            jax.ShapeDtypeStruct((N, SD), jnp.float32),
            jax.ShapeDtypeStruct((N, SD), jnp.float32),
            jax.ShapeDtypeStruct((1, NG), jnp.float32),
        ),
        in_specs=[
            pl.BlockSpec(memory_space=pltpu.SMEM),
            pl.BlockSpec((BN, 30), lambda i: (i, 0)),
            pl.BlockSpec((BN, 1), lambda i: (i, 0)),
            full((30, ED)), full((1, ED)), full((ED, ED)), full((1, ED)),
            full((30, SD)), full((1, SD)), full((30, SD)), full((30, SD)),
            full((1, SD)), full((ED, ED)), full((ED, ED)),
        ],
        out_specs=(
            pl.BlockSpec((BN, ED), lambda i: (i, 0)),
            pl.BlockSpec((BN, ED), lambda i: (i, 0)),
            pl.BlockSpec((BN, ED), lambda i: (i, 0)),
            pl.BlockSpec((BN, SD), lambda i: (i, 0)),
            pl.BlockSpec((BN, SD), lambda i: (i, 0)),
            pl.BlockSpec((BN, SD), lambda i: (i, 0)),
            pl.BlockSpec((1, NG), lambda i: (0, 0)),
        ),
    )(posmax, inp30, batch2d, ew1t, eb1, ew2t, eb2, wsd, bsd, wsn, wsu,
      bsu, whd0t, whs0t)


def _edge_mlp(pre, w2t, b2):
    nch = pre.shape[0]
    bc = nch // 32
    be = bc * R

    def body(pre_ref, w2t_ref, b2_ref, m_ref):
        sv = _silu(pre_ref[...].reshape(be, ED))
        mv = _silu(jnp.dot(sv.astype(jnp.bfloat16),
                           w2t_ref[...].astype(jnp.bfloat16),
                           preferred_element_type=jnp.float32)
                   + b2_ref[...])
        m_ref[...] = mv.reshape(bc, R, ED)

    return pl.pallas_call(
        body,
        grid=(nch // bc,),
        out_shape=jax.ShapeDtypeStruct((nch, R, ED), jnp.float32),
        in_specs=[
            pl.BlockSpec((bc, R, ED), lambda i: (i, 0, 0)),
            pl.BlockSpec((ED, ED), lambda i: (0, 0)),
            pl.BlockSpec((1, ED), lambda i: (0, 0)),
        ],
        out_specs=pl.BlockSpec((bc, R, ED), lambda i: (i, 0, 0)),
    )(pre, w2t, b2)


def _upd_body(h_ref, p_ref, deg_ref, su_ref, uht_ref, uat_ref, u2t_ref,
              ub2_ref, batch_ref, hn_ref, gsum_ref, gsq_ref):
    psum = p_ref[0] + p_ref[1]
    degv = jnp.maximum(deg_ref[0, :, 0:1] + deg_ref[1, :, 0:1], 1.0)
    agg = psum * (1.0 / degv)
    bf = jnp.bfloat16
    u1 = _silu(jnp.dot(h_ref[...].astype(bf), uht_ref[...].astype(bf),
                       preferred_element_type=jnp.float32)
               + jnp.dot(agg.astype(bf), uat_ref[...].astype(bf),
                         preferred_element_type=jnp.float32)
               + su_ref[...])
    up = _silu(jnp.dot(u1.astype(bf), u2t_ref[...].astype(bf),
                       preferred_element_type=jnp.float32) + ub2_ref[...])
    hn = h_ref[...] + up
    hn_ref[...] = hn
    mask = (batch_ref[...] ==
            lax.broadcasted_iota(jnp.int32, (1, NG), 1)).astype(jnp.float32)
    gs = lax.dot_general(mask, hn, (((0,), (0,)), ((), ())),
                         preferred_element_type=jnp.float32)
    gq = lax.dot_general(mask, hn * hn, (((0,), (0,)), ((), ())),
                         preferred_element_type=jnp.float32)

    @pl.when(pl.program_id(0) == 0)
    def _():
        gsum_ref[...] = jnp.zeros_like(gsum_ref)
        gsq_ref[...] = jnp.zeros_like(gsq_ref)

    gsum_ref[...] += gs
    gsq_ref[...] += gq


def _update(h, partials, deg2, su_all, layer, uht, uat, u2t, ub2, batch2d):
    return pl.pallas_call(
        _upd_body,
        grid=(GN,),
        out_shape=(
            jax.ShapeDtypeStruct((N, ED), jnp.float32),
            jax.ShapeDtypeStruct((NG, ED), jnp.float32),
            jax.ShapeDtypeStruct((NG, ED), jnp.float32),
        ),
        in_specs=[
            pl.BlockSpec((BN, ED), lambda i: (i, 0)),
            pl.BlockSpec((NC, BN, ED), lambda i: (0, i, 0)),
            pl.BlockSpec((NC, BN, ED), lambda i: (0, i, 0)),
            pl.BlockSpec((BN, ED), lambda i, L=layer: (i, L)),
            pl.BlockSpec((ED, ED), lambda i: (0, 0)),
            pl.BlockSpec((ED, ED), lambda i: (0, 0)),
            pl.BlockSpec((ED, ED), lambda i: (0, 0)),
            pl.BlockSpec((1, ED), lambda i: (0, 0)),
            pl.BlockSpec((BN, 1), lambda i: (i, 0)),
        ],
        out_specs=(
            pl.BlockSpec((BN, ED), lambda i: (i, 0)),
            pl.BlockSpec((NG, ED), lambda i: (0, 0)),
            pl.BlockSpec((NG, ED), lambda i: (0, 0)),
        ),
    )(h, partials, deg2, su_all, uht, uat, u2t, ub2, batch2d)


def _norm_body_proj(hn_ref, gsum_ref, gsq_ref, cb_ref, batch_ref, whdt_ref,
                    whst_ref, sd_ref, ssn_ref, h_ref, a_ref, b_ref):
    hb = _norm_common(hn_ref, gsum_ref, gsq_ref, cb_ref, batch_ref)
    h_ref[...] = hb
    a_ref[...] = jnp.dot(hb, whdt_ref[...],
                         preferred_element_type=jnp.float32) + sd_ref[...]
    b_ref[...] = jnp.dot(hb, whst_ref[...],
                         preferred_element_type=jnp.float32) + ssn_ref[...]


def _norm_body_last(hn_ref, gsum_ref, gsq_ref, cb_ref, batch_ref, h_ref):
    h_ref[...] = _norm_common(hn_ref, gsum_ref, gsq_ref, cb_ref, batch_ref)


def _norm_common(hn_ref, gsum_ref, gsq_ref, cb_ref, batch_ref):
    cbm = jnp.maximum(cb_ref[...], 1.0)
    mask = (batch_ref[...] ==
            lax.broadcasted_iota(jnp.int32, (1, NG), 1)).astype(jnp.float32)
    maskc = mask * (1.0 / cbm)
    meanr = jnp.dot(maskc, gsum_ref[...], preferred_element_type=jnp.float32)
    eh2r = jnp.dot(maskc, gsq_ref[...], preferred_element_type=jnp.float32)
    varr = jnp.maximum(eh2r - meanr * meanr, 0.0)
    return (hn_ref[...] - meanr) * lax.rsqrt(varr + EPS)


def _norm_proj(hn, gsum, gsq, cb, batch2d, whdt, whst, sd_all, ssn_all,
               layer_next):
    return pl.pallas_call(
        _norm_body_proj,
        grid=(GN,),
        out_shape=(
            jax.ShapeDtypeStruct((N, ED), jnp.float32),
            jax.ShapeDtypeStruct((N, ED), jnp.float32),
            jax.ShapeDtypeStruct((N, ED), jnp.float32),
        ),
        in_specs=[
            pl.BlockSpec((BN, ED), lambda i: (i, 0)),
            pl.BlockSpec((NG, ED), lambda i: (0, 0)),
            pl.BlockSpec((NG, ED), lambda i: (0, 0)),
            pl.BlockSpec((1, NG), lambda i: (0, 0)),
            pl.BlockSpec((BN, 1), lambda i: (i, 0)),
            pl.BlockSpec((ED, ED), lambda i: (0, 0)),
            pl.BlockSpec((ED, ED), lambda i: (0, 0)),
            pl.BlockSpec((BN, ED), lambda i, L=layer_next: (i, L)),
            pl.BlockSpec((BN, ED), lambda i, L=layer_next: (i, L)),
        ],
        out_specs=(
            pl.BlockSpec((BN, ED), lambda i: (i, 0)),
            pl.BlockSpec((BN, ED), lambda i: (i, 0)),
            pl.BlockSpec((BN, ED), lambda i: (i, 0)),
        ),
    )(hn, gsum, gsq, cb, batch2d, whdt, whst, sd_all, ssn_all)


def _norm_last(hn, gsum, gsq, cb, batch2d):
    return pl.pallas_call(
        _norm_body_last,
        grid=(GN,),
        out_shape=jax.ShapeDtypeStruct((N, ED), jnp.float32),
        in_specs=[
            pl.BlockSpec((BN, ED), lambda i: (i, 0)),
            pl.BlockSpec((NG, ED), lambda i: (0, 0)),
            pl.BlockSpec((NG, ED), lambda i: (0, 0)),
            pl.BlockSpec((1, NG), lambda i: (0, 0)),
            pl.BlockSpec((BN, 1), lambda i: (i, 0)),
        ],
        out_specs=pl.BlockSpec((BN, ED), lambda i: (i, 0)),
    )(hn, gsum, gsq, cb, batch2d)


def _dec_body(h_ref, x_ref, dt_ref, w1d_ref, b1d_ref, w2d_ref, b2s_ref,
              out_ref):
    z1 = _silu(jnp.dot(h_ref[...], w1d_ref[...],
                       preferred_element_type=jnp.float32) + b1d_ref[...])
    z2 = jnp.dot(z1, w2d_ref[...],
                 preferred_element_type=jnp.float32) + b2s_ref[0, 0]
    steps = (lax.broadcasted_iota(jnp.int32, (1, TW), 1) + 1
             ).astype(jnp.float32)
    dtv = dt_ref[0, 0] * steps
    out_ref[...] = x_ref[:, TW - 1:TW] + dtv * z2


def _decoder(h, x, dt2d, w1d, b1d, w2d, b2s):
    return pl.pallas_call(
        _dec_body,
        grid=(GN,),
        out_shape=jax.ShapeDtypeStruct((N, TW), jnp.float32),
        in_specs=[
            pl.BlockSpec((BN, ED), lambda i: (i, 0)),
            pl.BlockSpec((BN, TW), lambda i: (i, 0)),
            pl.BlockSpec(memory_space=pltpu.SMEM),
            pl.BlockSpec((ED, 8 * 38), lambda i: (0, 0)),
            pl.BlockSpec((1, 8 * 38), lambda i: (0, 0)),
            pl.BlockSpec((8 * 38, TW), lambda i: (0, 0)),
            pl.BlockSpec(memory_space=pltpu.SMEM),
        ],
        out_specs=pl.BlockSpec((BN, TW), lambda i: (i, 0)),
    )(h, x, dt2d, w1d, b1d, w2d, b2s)


def _densify_convs(conv1_w, conv1_b, conv2_w, conv2_b):
    o_i, p_i, k_i = np.meshgrid(np.arange(8), np.arange(38), np.arange(16),
                                indexing="ij")
    rows1 = (3 * p_i + k_i).reshape(-1)
    cols1 = (o_i * 38 + p_i).reshape(-1)
    w1d = jnp.zeros((ED, 8 * 38), jnp.float32).at[rows1, cols1].set(
        conv1_w[o_i.reshape(-1), 0, k_i.reshape(-1)])
    b1d = jnp.repeat(conv1_b, 38).reshape(1, 8 * 38)
    o_i, q_i, d_i = np.meshgrid(np.arange(8), np.arange(TW), np.arange(14),
                                indexing="ij")
    rows2 = (o_i * 38 + q_i + d_i).reshape(-1)
    cols2 = q_i.reshape(-1)
    w2d = jnp.zeros((8 * 38, TW), jnp.float32).at[rows2, cols2].set(
        conv2_w[0, o_i.reshape(-1), d_i.reshape(-1)])
    b2s = conv2_b.reshape(1, 1)
    return w1d, b1d, w2d, b2s


def kernel(x, pos, t, vars_abc, dt, enc_w1, enc_b1, enc_w2, enc_b2,
           msg1_w, msg1_b, msg2_w, msg2_b, upd1_w, upd1_b, upd2_w, upd2_b,
           conv1_w, conv1_b, conv2_w, conv2_b, edge_index, batch):
    f32 = jnp.float32
    SD = NL * ED

    inp30 = jnp.concatenate([x, pos, t, vars_abc], axis=-1)
    batch2d = batch.reshape(N, 1).astype(jnp.int32)
    srcv = edge_index[0].astype(jnp.int32)
    dstv = edge_index[1].astype(jnp.int32)
    dst3d = dstv.reshape(NW, J, R)
    jsl = (41, 41, 43)
    esl = [NW * jc * R for jc in jsl]
    o1, o2 = esl[0], esl[0] + esl[1]
    srcS = (srcv[:o1].reshape(NW, jsl[0], R),
            srcv[o1:o2].reshape(NW, jsl[1], R),
            srcv[o2:].reshape(NW, jsl[2], R))
    dstS = (dstv[:o1].reshape(NW, jsl[0], R),
            dstv[o1:o2].reshape(NW, jsl[1], R),
            dstv[o2:].reshape(NW, jsl[2], R))
    pos_pad = jnp.concatenate(
        [pos.reshape(-1), jnp.zeros((240,), f32)]).reshape(80, 128)

    whd_t = jnp.transpose(msg1_w[:, :, :ED], (0, 2, 1))
    whs_t = jnp.transpose(msg1_w[:, :, ED:2 * ED], (0, 2, 1))
    wx = msg1_w[:, :, 2 * ED:2 * ED + TW]
    wp = msg1_w[:, :, 2 * ED + TW:2 * ED + TW + 1]
    wv = msg1_w[:, :, 2 * ED + TW + 1:]
    wsd = jnp.concatenate([wx, wp, wv], axis=2)
    wsd_t = wsd.reshape(SD, 30).T
    bsd = msg1_b.reshape(1, SD)
    wsn = -jnp.concatenate([wx, wp, jnp.zeros((NL, ED, 4), f32)], axis=2)
    wsn_t = wsn.reshape(SD, 30).T
    uv = upd1_w[:, :, 2 * ED:]
    wsu = jnp.concatenate([jnp.zeros((NL, ED, 26), f32), uv], axis=2)
    wsu_t = wsu.reshape(SD, 30).T
    bsu = upd1_b.reshape(1, SD)
    uh_t = jnp.transpose(upd1_w[:, :, :ED], (0, 2, 1))
    ua_t = jnp.transpose(upd1_w[:, :, ED:2 * ED], (0, 2, 1))
    w2_t = jnp.transpose(msg2_w, (0, 2, 1))
    u2_t = jnp.transpose(upd2_w, (0, 2, 1))
    ew1t = enc_w1.T
    ew2t = enc_w2.T
    eb1 = enc_b1.reshape(1, ED)
    eb2 = enc_b2.reshape(1, ED)
    mb2 = msg2_b.reshape(NL, 1, ED)
    ub2 = upd2_b.reshape(NL, 1, ED)
    w1d, b1d, w2d, b2s = _densify_convs(conv1_w, conv1_b, conv2_w, conv2_b)
    dt2d = dt.reshape(1, 1)

    posmax = _posmax(pos_pad)
    h, a, b, sd_all, ssn_all, su_all, cb = _encoder(
        posmax, inp30, batch2d, ew1t, eb1, ew2t, eb2, wsd_t, bsd, wsn_t,
        wsu_t, bsu, whd_t[0].copy(), whs_t[0].copy())
    deg2 = _sc_deg(dst3d)

    for l in range(NL):
        pre1 = _sc_gather(a, b, dstS[0], srcS[0])
        pre2 = _sc_gather(a, b, dstS[1], srcS[1])
        m1 = _edge_mlp(pre1, w2_t[l], mb2[l])
        pre3 = _sc_gather(a, b, dstS[2], srcS[2])
        m2 = _edge_mlp(pre2, w2_t[l], mb2[l])
        m3 = _edge_mlp(pre3, w2_t[l], mb2[l])
        partials = _sc_scatter3(m1, m2, m3, dstS[0], dstS[1], dstS[2])
        hn, gsum, gsq = _update(h, partials, deg2, su_all, l, uh_t[l],
                                ua_t[l], u2_t[l], ub2[l], batch2d)
        if l < NL - 1:
            h, a, b = _norm_proj(hn, gsum, gsq, cb, batch2d, whd_t[l + 1],
                                 whs_t[l + 1], sd_all, ssn_all, l + 1)
        else:
            h = _norm_last(hn, gsum, gsq, cb, batch2d)

    return _decoder(h, x, dt2d, w1d, b1d, w2d, b2s)
